# baseline (speedup 1.0000x reference)
import functools

import jax
import jax.numpy as jnp
from jax import lax
from jax.experimental import pallas as pl
from jax.experimental.pallas import tpu as pltpu
from jax.experimental.pallas import tpu_sc as plsc

G = 16
N = 4096
E = 131072
L = 128

NBLK = 8
BN = N // NBLK
EBLK = 128
BE = E // EBLK

_NEG = -1e9


def _leaky(v):
    return jnp.where(v >= 0, v, 0.01 * v)


_NC, _NS = 2, 16
_SC_PARAMS = pltpu.CompilerParams(needs_layout_passes=False)
_NW = _NC * _NS
_EPW = E // _NW
_GCH = 512
_NCH = _EPW // _GCH


def _sc_gather2_body(a_hbm, b_hbm, dst_hbm, src_hbm, adst_hbm, bsrc_hbm,
                     idx_v, rows_v, sem):
    wid = lax.axis_index("s") * _NC + lax.axis_index("c")

    def chunk(i, _):
        base = wid * _EPW + i * _GCH
        pltpu.sync_copy(dst_hbm.at[pl.ds(base, _GCH)], idx_v)
        pltpu.async_copy(a_hbm.at[idx_v], rows_v, sem).wait()
        pltpu.sync_copy(rows_v, adst_hbm.at[pl.ds(base, _GCH)])
        pltpu.sync_copy(src_hbm.at[pl.ds(base, _GCH)], idx_v)
        pltpu.async_copy(b_hbm.at[idx_v], rows_v, sem).wait()
        pltpu.sync_copy(rows_v, bsrc_hbm.at[pl.ds(base, _GCH)])
        return ()

    lax.fori_loop(0, _NCH, chunk, (), unroll=False)


def _sc_gather2(a, b, dst, src):
    mesh = plsc.VectorSubcoreMesh(core_axis_name="c", subcore_axis_name="s")
    f = pl.kernel(
        _sc_gather2_body,
        mesh=mesh,
        compiler_params=_SC_PARAMS,
        out_type=(
            jax.ShapeDtypeStruct((E, L), jnp.float32),
            jax.ShapeDtypeStruct((E, L), jnp.float32),
        ),
        scratch_types=[
            pltpu.VMEM((_GCH,), jnp.int32),
            pltpu.VMEM((_GCH, L), jnp.float32),
            pltpu.SemaphoreType.DMA,
        ],
    )
    return f(a, b, dst, src)


CSEG = 256
_FB = 1024
ECAP = E + CSEG
_NPW = N // _NW
_DUMP = _NPW * 8
PREDPAD = 16 * _NW


def _append_flush(buf_refs, hbm_refs, vals, mask, cnt, nf, wbase):
    for br, v in zip(buf_refs, vals):
        plsc.store_compressed(br.at[pl.ds(cnt, 16)], v, mask=mask)
    cnt = cnt + jnp.sum(mask.astype(jnp.int32))

    def flush():
        for br, hr in zip(buf_refs, hbm_refs):
            pltpu.sync_copy(br.at[pl.ds(0, _FB)],
                            hr.at[pl.ds(wbase + nf * _FB, _FB)])
            rem = br[pl.ds(_FB, 16)]
            br[pl.ds(0, 16)] = rem

    jax.lax.cond(cnt >= _FB, flush, lambda: None)
    new_nf = jnp.where(cnt >= _FB, nf + 1, nf)
    new_cnt = jnp.where(cnt >= _FB, cnt - _FB, cnt)
    return new_cnt, new_nf


def _pad_tail(buf_refs, hbm_refs, pads, cnt, nf, wbase):
    base16 = (cnt // 16) * 16
    lanes = lax.iota(jnp.int32, 16)
    for br, padv in zip(buf_refs, pads):
        cur = br[pl.ds(base16, 16)]
        br[pl.ds(base16, 16)] = jnp.where(base16 + lanes < cnt, cur, padv)
        for k in range(1, 18):
            br[pl.ds(base16 + k * 16, 16)] = jnp.zeros((16,), jnp.int32) + padv
    padded = ((cnt + CSEG - 1) // CSEG) * CSEG

    def flush_k(k, _):
        for br, hr in zip(buf_refs, hbm_refs):
            pltpu.sync_copy(br.at[pl.ds(k * CSEG, CSEG)],
                            hr.at[pl.ds(wbase + nf * _FB + k * CSEG, CSEG)])
        return ()

    lax.fori_loop(0, padded // CSEG, flush_k, ())
    return nf * _FB + padded


_PCH = 4096


def _sc_prep_body(dst_hbm, src_hbm,
                  did_hbm, dloc_hbm, dcnt_hbm, addr_hbm, spid_hbm, scnt_hbm,
                  d_v, s_v, did_v, dloc_v, addr_v, spid_v, cnt_v):
    wid = lax.axis_index("s") * _NC + lax.axis_index("c")
    lo = wid * _NPW
    wbase = wid * ECAP

    def chunk(i, carry):
        cnt1, nf1, cnt2, nf2 = carry
        pltpu.sync_copy(dst_hbm.at[pl.ds(i * _PCH, _PCH)], d_v)
        pltpu.sync_copy(src_hbm.at[pl.ds(i * _PCH, _PCH)], s_v)

        def vreg(j, carry2):
            c1, n1, c2, n2 = carry2
            d = d_v[pl.ds(j * 16, 16)]
            s = s_v[pl.ds(j * 16, 16)]
            ids = lax.iota(jnp.int32, 16) + (i * _PCH + j * 16)
            mask_d = (d >= lo) & (d < lo + _NPW)
            c1, n1 = _append_flush((did_v, dloc_v), (did_hbm, dloc_hbm),
                                   (ids, d - lo), mask_d, c1, n1, wbase)
            mask_s = (s >= lo) & (s < lo + _NPW)
            addr = s * N + d
            c2, n2 = _append_flush((addr_v, spid_v), (addr_hbm, spid_hbm),
                                   (addr, ids), mask_s, c2, n2, wbase)
            return c1, n1, c2, n2

        return lax.fori_loop(0, _PCH // 16, vreg, (cnt1, nf1, cnt2, nf2))

    cnt1, nf1, cnt2, nf2 = lax.fori_loop(
        0, E // _PCH, chunk,
        (jnp.int32(0), jnp.int32(0), jnp.int32(0), jnp.int32(0)))

    tot1 = _pad_tail((did_v, dloc_v), (did_hbm, dloc_hbm),
                     (jnp.int32(0), jnp.int32(_DUMP) // 8), cnt1, nf1, wbase)
    tot2 = _pad_tail((addr_v, spid_v), (addr_hbm, spid_hbm),
                     (jnp.int32(N * N) + wid * 16, jnp.int32(0)),
                     cnt2, nf2, wbase)
    cnt_v[...] = jnp.zeros((16,), jnp.int32) + tot1
    pltpu.sync_copy(cnt_v, dcnt_hbm.at[pl.ds(wid * 16, 16)])
    cnt_v[...] = jnp.zeros((16,), jnp.int32) + tot2
    pltpu.sync_copy(cnt_v, scnt_hbm.at[pl.ds(wid * 16, 16)])


def _sc_prep(dst, src):
    mesh = plsc.VectorSubcoreMesh(core_axis_name="c", subcore_axis_name="s")
    lbuf = pltpu.VMEM((_FB + 16 + 288,), jnp.int32)
    f = pl.kernel(
        _sc_prep_body,
        mesh=mesh,
        compiler_params=_SC_PARAMS,
        out_type=(
            jax.ShapeDtypeStruct((_NW * ECAP,), jnp.int32),
            jax.ShapeDtypeStruct((_NW * ECAP,), jnp.int32),
            jax.ShapeDtypeStruct((_NW * 16,), jnp.int32),
            jax.ShapeDtypeStruct((_NW * ECAP,), jnp.int32),
            jax.ShapeDtypeStruct((_NW * ECAP,), jnp.int32),
            jax.ShapeDtypeStruct((_NW * 16,), jnp.int32),
        ),
        scratch_types=[
            pltpu.VMEM((_PCH,), jnp.int32),
            pltpu.VMEM((_PCH,), jnp.int32),
            lbuf, lbuf, lbuf, lbuf,
            pltpu.VMEM((16,), jnp.int32),
        ],
    )
    return f(dst, src)


def _sc_segmax_body(m_hbm, did_hbm, dloc_hbm, dcnt_hbm, aggr_hbm,
                    ids_v, dl_v, rows_v, acc0_v, acc1_v, acc2_v, acc3_v,
                    cnt_v, sem):
    accs = (acc0_v, acc1_v, acc2_v, acc3_v)
    wid = lax.axis_index("s") * _NC + lax.axis_index("c")
    wbase = wid * ECAP

    def initrow(j, _):
        for a in accs:
            for c in range(8):
                a[j, pl.ds(c * 16, 16)] = jnp.full((16,), -jnp.inf, jnp.float32)
        return ()

    lax.fori_loop(0, _NPW + 1, initrow, ())

    pltpu.sync_copy(dcnt_hbm.at[pl.ds(wid * 16, 16)], cnt_v)
    trips = cnt_v[...][0] // CSEG

    def chunk(t, _):
        pltpu.sync_copy(did_hbm.at[pl.ds(wbase + t * CSEG, CSEG)], ids_v)
        pltpu.sync_copy(dloc_hbm.at[pl.ds(wbase + t * CSEG, CSEG)], dl_v)
        pltpu.async_copy(m_hbm.at[ids_v], rows_v, sem).wait()

        def group(g, _):
            dlv = dl_v[pl.ds(g * 16, 16)]
            for k in range(16):
                e = g * 16 + k
                acc = accs[k % 4]
                dl = dlv[k]
                for c in range(8):
                    r = rows_v[e, pl.ds(c * 16, 16)]
                    acc[dl, pl.ds(c * 16, 16)] = jnp.maximum(
                        acc[dl, pl.ds(c * 16, 16)], r)
            return ()

        lax.fori_loop(0, CSEG // 16, group, ())
        return ()

    lax.fori_loop(0, trips, chunk, ())

    def mergerow(j, _):
        for c in range(8):
            m01 = jnp.maximum(acc0_v[j, pl.ds(c * 16, 16)],
                              acc1_v[j, pl.ds(c * 16, 16)])
            m23 = jnp.maximum(acc2_v[j, pl.ds(c * 16, 16)],
                              acc3_v[j, pl.ds(c * 16, 16)])
            acc0_v[j, pl.ds(c * 16, 16)] = jnp.maximum(m01, m23)
        return ()

    lax.fori_loop(0, _NPW, mergerow, ())
    pltpu.sync_copy(acc0_v.at[pl.ds(0, _NPW)],
                    aggr_hbm.at[pl.ds(wid * _NPW, _NPW)])


def _sc_segmax(m, did, dloc, dcnt):
    mesh = plsc.VectorSubcoreMesh(core_axis_name="c", subcore_axis_name="s")
    f = pl.kernel(
        _sc_segmax_body,
        mesh=mesh,
        compiler_params=_SC_PARAMS,
        out_type=jax.ShapeDtypeStruct((N, L), jnp.float32),
        scratch_types=[
            pltpu.VMEM((CSEG,), jnp.int32),
            pltpu.VMEM((CSEG,), jnp.int32),
            pltpu.VMEM((CSEG, L), jnp.float32),
            pltpu.VMEM((_NPW + 1, L), jnp.float32),
            pltpu.VMEM((_NPW + 1, L), jnp.float32),
            pltpu.VMEM((_NPW + 1, L), jnp.float32),
            pltpu.VMEM((_NPW + 1, L), jnp.float32),
            pltpu.VMEM((16,), jnp.int32),
            pltpu.SemaphoreType.DMA,
        ],
    )
    return f(m, did, dloc, dcnt)


_FILLB = 16384
_RPW = N * N // _NW


def _sc_pred_body(pout_hbm, addr_hbm, spid_hbm, scnt_hbm, pred_hbm,
                  fill_v, addr_v, pid_v, vals_v, cnt_v, sem):
    wid = lax.axis_index("s") * _NC + lax.axis_index("c")
    wbase = wid * ECAP

    def initf(j, _):
        fill_v[pl.ds(j * 16, 16)] = jnp.full((16,), _NEG, jnp.float32)
        return ()

    lax.fori_loop(0, _FILLB // 16, initf, ())

    def fill(k, _):
        pltpu.sync_copy(fill_v,
                        pred_hbm.at[pl.ds(wid * _RPW + k * _FILLB, _FILLB)])
        return ()

    lax.fori_loop(0, _RPW // _FILLB, fill, ())

    pltpu.sync_copy(scnt_hbm.at[pl.ds(wid * 16, 16)], cnt_v)
    trips = cnt_v[...][0] // CSEG

    def chunk(t, _):
        pltpu.sync_copy(addr_hbm.at[pl.ds(wbase + t * CSEG, CSEG)], addr_v)
        pltpu.sync_copy(spid_hbm.at[pl.ds(wbase + t * CSEG, CSEG)], pid_v)
        pltpu.async_copy(pout_hbm.at[pid_v], vals_v, sem).wait()
        pltpu.async_copy(vals_v, pred_hbm.at[addr_v], sem).wait()
        return ()

    lax.fori_loop(0, trips, chunk, ())


def _sc_pred(pout, addr, spid, scnt):
    mesh = plsc.VectorSubcoreMesh(core_axis_name="c", subcore_axis_name="s")
    f = pl.kernel(
        _sc_pred_body,
        mesh=mesh,
        compiler_params=_SC_PARAMS,
        out_type=jax.ShapeDtypeStruct((N * N + PREDPAD,), jnp.float32),
        scratch_types=[
            pltpu.VMEM((_FILLB,), jnp.float32),
            pltpu.VMEM((CSEG,), jnp.int32),
            pltpu.VMEM((CSEG,), jnp.int32),
            pltpu.VMEM((CSEG,), jnp.float32),
            pltpu.VMEM((16,), jnp.int32),
            pltpu.SemaphoreType.DMA,
        ],
    )
    return f(pout, addr, spid, scnt)


def _prep_body(pt_ref, h_ref, w0_ref, wenc1_ref, wm1d_ref, wm1s_ref,
               enc_ref, a_ref, b_ref):
    enc = jax.nn.relu(pt_ref[...] * w0_ref[...] +
                      jnp.dot(h_ref[...], wenc1_ref[...],
                              preferred_element_type=jnp.float32))
    enc_ref[...] = enc
    a_ref[...] = jnp.dot(enc, wm1d_ref[...], preferred_element_type=jnp.float32)
    b_ref[...] = jnp.dot(enc, wm1s_ref[...], preferred_element_type=jnp.float32)


def _prep(pt, h, w0, wenc1, wm1d, wm1s):
    row = pl.BlockSpec((BN, L), lambda i: (i, 0))
    col = pl.BlockSpec((BN, 1), lambda i: (i, 0))
    full = pl.BlockSpec((L, L), lambda i: (0, 0))
    vec = pl.BlockSpec((1, L), lambda i: (0, 0))
    return pl.pallas_call(
        _prep_body,
        grid=(NBLK,),
        in_specs=[col, row, vec, full, full, full],
        out_specs=[row, row, row],
        out_shape=[jax.ShapeDtypeStruct((N, L), jnp.float32)] * 3,
    )(pt, h, w0, wenc1, wm1d, wm1s)


def _mmlp_body(a_ref, b_ref, ea_ref, w_ref, wm2_ref, o_ref):
    pre = a_ref[...] + b_ref[...] + ea_ref[...] * w_ref[...]
    q = _leaky(pre)
    o_ref[...] = _leaky(jnp.dot(q, wm2_ref[...],
                                preferred_element_type=jnp.float32))


def _mmlp(adst, bsrc, ea2, w, wm2):
    row = pl.BlockSpec((BE, L), lambda i: (i, 0))
    col = pl.BlockSpec((BE, 1), lambda i: (i, 0))
    full = pl.BlockSpec((L, L), lambda i: (0, 0))
    vec = pl.BlockSpec((1, L), lambda i: (0, 0))
    return pl.pallas_call(
        _mmlp_body,
        grid=(EBLK,),
        in_specs=[row, row, col, vec, full],
        out_specs=row,
        out_shape=jax.ShapeDtypeStruct((E, L), jnp.float32),
    )(adst, bsrc, ea2, w, wm2)


def _gru(u, gh_ref, h, w_iht_ref):
    gi = jnp.dot(u, w_iht_ref[...], preferred_element_type=jnp.float32)
    gh = gh_ref
    r = jax.nn.sigmoid(gi[:, :L] + gh[:, :L])
    z = jax.nn.sigmoid(gi[:, L:2 * L] + gh[:, L:2 * L])
    ng = jnp.tanh(gi[:, 2 * L:] + r * gh[:, 2 * L:])
    return (1.0 - z) * ng + z * h


def _update_body(enc_ref, raw_ref, h_ref, wue_ref, wua_ref, wiht_ref,
                 whht_ref, wmste_ref, wmsth_ref, hn_ref, mst_ref):
    raw = raw_ref[...]
    aggr = jnp.where(raw > -jnp.inf, raw, 0.0)
    enc = enc_ref[...]
    h = h_ref[...]
    u = _leaky(jnp.dot(enc, wue_ref[...], preferred_element_type=jnp.float32) +
               jnp.dot(aggr, wua_ref[...], preferred_element_type=jnp.float32))
    gh = jnp.dot(h, whht_ref[...], preferred_element_type=jnp.float32)
    hn = _gru(u, gh, h, wiht_ref)
    hn_ref[...] = hn
    mst_ref[...] = (jnp.dot(enc, wmste_ref[...], preferred_element_type=jnp.float32) +
                    jnp.dot(hn, wmsth_ref[...], preferred_element_type=jnp.float32))


def _update(enc, raw, h, wue, wua, wiht, whht, wmste, wmsth):
    row = pl.BlockSpec((BN, L), lambda i: (i, 0))
    full = pl.BlockSpec((L, L), lambda i: (0, 0))
    full3 = pl.BlockSpec((L, 3 * L), lambda i: (0, 0))
    cvec = pl.BlockSpec((L, 1), lambda i: (0, 0))
    col = pl.BlockSpec((BN, 1), lambda i: (i, 0))
    return pl.pallas_call(
        _update_body,
        grid=(NBLK,),
        in_specs=[row, row, row, full, full, full3, full3, cvec, cvec],
        out_specs=[row, col],
        out_shape=[jax.ShapeDtypeStruct((N, L), jnp.float32),
                   jax.ShapeDtypeStruct((N, 1), jnp.float32)],
    )(enc, raw, h, wue, wua, wiht, whht, wmste, wmsth)


def _update0_body(emax_ref, emin_ref, wm1e_ref, wm2_ref, wua_ref, wiht_ref,
                  wmsth_ref, hn_ref, mst_ref):
    g = _leaky(jnp.dot(_leaky(wm1e_ref[...]), wm2_ref[...],
                       preferred_element_type=jnp.float32))
    emax_raw = emax_ref[...]
    emin_raw = emin_ref[...]
    emax = jnp.where(emax_raw > -jnp.inf, emax_raw, 0.0)
    emin = jnp.where(emin_raw < jnp.inf, emin_raw, 0.0)
    aggr = jnp.where(g > 0, emax * g, emin * g)
    u = _leaky(jnp.dot(aggr, wua_ref[...], preferred_element_type=jnp.float32))
    gi = jnp.dot(u, wiht_ref[...], preferred_element_type=jnp.float32)
    z = jax.nn.sigmoid(gi[:, L:2 * L])
    ng = jnp.tanh(gi[:, 2 * L:])
    hn = (1.0 - z) * ng
    hn_ref[...] = hn
    mst_ref[...] = jnp.dot(hn, wmsth_ref[...], preferred_element_type=jnp.float32)


def _update0(emax, emin, wm1e, wm2, wua, wiht, wmsth):
    row = pl.BlockSpec((BN, L), lambda i: (i, 0))
    col = pl.BlockSpec((BN, 1), lambda i: (i, 0))
    full = pl.BlockSpec((L, L), lambda i: (0, 0))
    full3 = pl.BlockSpec((L, 3 * L), lambda i: (0, 0))
    vec = pl.BlockSpec((1, L), lambda i: (0, 0))
    cvec = pl.BlockSpec((L, 1), lambda i: (0, 0))
    return pl.pallas_call(
        _update0_body,
        grid=(NBLK,),
        in_specs=[col, col, vec, full, full, full3, cvec],
        out_specs=[row, col],
        out_shape=[jax.ShapeDtypeStruct((N, L), jnp.float32),
                   jax.ShapeDtypeStruct((N, 1), jnp.float32)],
    )(emax, emin, wm1e, wm2, wua, wiht, wmsth)


def _select_body(mst_ref, pt_ref, out_ref):
    mst = mst_ref[...]
    pt = pt_ref[...]
    nt = jnp.where(pt != 0, _NEG, mst)
    iota = lax.broadcasted_iota(jnp.int32, (G, N // G), 1)
    rowmax = jnp.max(nt, axis=1, keepdims=True)
    cand = jnp.where(nt == rowmax, iota, jnp.int32(2**30))
    chosen = jnp.min(cand, axis=1, keepdims=True)
    out_ref[...] = jnp.where(iota == chosen, 1.0, pt)


def _select(mst_g, pt_g):
    blk = pl.BlockSpec((G, N // G), lambda: (0, 0))
    return pl.pallas_call(
        _select_body,
        in_specs=[blk, blk],
        out_specs=blk,
        out_shape=jax.ShapeDtypeStruct((G, N // G), jnp.float32),
    )(mst_g, pt_g)


def _final_body(enc_ref, raw_ref, h_ref, wue_ref, wua_ref, wiht_ref,
                whht_ref, wp1s_ref, wp1d_ref, c_ref, d_ref):
    raw = raw_ref[...]
    aggr = jnp.where(raw > -jnp.inf, raw, 0.0)
    enc = enc_ref[...]
    h = h_ref[...]
    u = _leaky(jnp.dot(enc, wue_ref[...], preferred_element_type=jnp.float32) +
               jnp.dot(aggr, wua_ref[...], preferred_element_type=jnp.float32))
    gh = jnp.dot(h, whht_ref[...], preferred_element_type=jnp.float32)
    hn = _gru(u, gh, h, wiht_ref)
    c_ref[...] = jnp.dot(hn, wp1s_ref[...], preferred_element_type=jnp.float32)
    d_ref[...] = jnp.dot(hn, wp1d_ref[...], preferred_element_type=jnp.float32)


def _final_update(enc, raw, h, wue, wua, wiht, whht, wp1s, wp1d):
    row = pl.BlockSpec((BN, L), lambda i: (i, 0))
    full = pl.BlockSpec((L, L), lambda i: (0, 0))
    full3 = pl.BlockSpec((L, 3 * L), lambda i: (0, 0))
    return pl.pallas_call(
        _final_body,
        grid=(NBLK,),
        in_specs=[row, row, row, full, full, full3, full3, full, full],
        out_specs=[row, row],
        out_shape=[jax.ShapeDtypeStruct((N, L), jnp.float32)] * 2,
    )(enc, raw, h, wue, wua, wiht, whht, wp1s, wp1d)


def _pout_body(c_ref, d_ref, ea_ref, w_ref, wp2_ref, o_ref):
    pin = c_ref[...] + d_ref[...] + ea_ref[...] * w_ref[...]
    o_ref[...] = jnp.dot(jax.nn.relu(pin), wp2_ref[...],
                         preferred_element_type=jnp.float32)


def _pout(csrc, ddst, ea2, wp1e, wp2):
    row = pl.BlockSpec((BE, L), lambda i: (i, 0))
    col = pl.BlockSpec((BE, 1), lambda i: (i, 0))
    vec = pl.BlockSpec((1, L), lambda i: (0, 0))
    cvec = pl.BlockSpec((L, 1), lambda i: (0, 0))
    return pl.pallas_call(
        _pout_body,
        grid=(EBLK,),
        in_specs=[row, row, col, vec, cvec],
        out_specs=col,
        out_shape=jax.ShapeDtypeStruct((E, 1), jnp.float32),
    )(csrc, ddst, ea2, wp1e, wp2)


def kernel(x, edge_attr, edge_index, W_enc, W_m1, W_m2, W_u, W_ih, W_hh, W_mst, W_p1, W_p2):
    n = x.shape[0]
    steps = x.shape[1]
    src = edge_index[0]
    dst = edge_index[1]
    ea = edge_attr
    ea2 = ea[:, None]

    w_enc0 = W_enc[0:1]
    W_enc1 = W_enc[1:]
    Wm1_d = W_m1[:L]
    Wm1_s = W_m1[L:2 * L]
    wm1_e = W_m1[2 * L:2 * L + 1]
    Wu_e = W_u[:L]
    Wu_a = W_u[L:]
    W_ihT = W_ih.T
    W_hhT = W_hh.T
    Wmst_e = W_mst[:L]
    Wmst_h = W_mst[L:]
    Wp1_s = W_p1[:L]
    Wp1_d = W_p1[L:2 * L]
    wp1_e = W_p1[2 * L:2 * L + 1]

    pt = x[:, 0]

    did, dloc, dcnt, addr, spid, scnt = _sc_prep(dst, src)

    ea_max = jax.ops.segment_max(ea, dst, num_segments=n)[:, None]
    ea_min = jax.ops.segment_min(ea, dst, num_segments=n)[:, None]
    h, mst = _update0(ea_max, ea_min, wm1_e, W_m2, Wu_a, W_ihT, Wmst_h)
    pt = _select(mst.reshape(G, n // G), pt.reshape(G, n // G)).reshape(-1)

    for step in range(1, steps):
        enc, A, B = _prep(pt[:, None], h, w_enc0, W_enc1, Wm1_d, Wm1_s)
        adst, bsrc = _sc_gather2(A, B, dst, src)
        m = _mmlp(adst, bsrc, ea2, wm1_e, W_m2)
        raw = _sc_segmax(m, did, dloc, dcnt)
        if step < steps - 1:
            h, mst = _update(enc, raw, h, Wu_e, Wu_a, W_ihT, W_hhT,
                             Wmst_e, Wmst_h)
            pt = _select(mst.reshape(G, n // G),
                         pt.reshape(G, n // G)).reshape(-1)
        else:
            C, D = _final_update(enc, raw, h, Wu_e, Wu_a, W_ihT, W_hhT,
                                 Wp1_s, Wp1_d)

    csrc, ddst = _sc_gather2(C, D, src, dst)
    p_out = _pout(csrc, ddst, ea2, wp1_e, W_p2)[:, 0]
    pred_flat = _sc_pred(p_out, addr, spid, scnt)
    return pred_flat[:n * n].reshape(n, n)


# speedup vs baseline: 1.0102x; 1.0102x over previous
import functools

import jax
import jax.numpy as jnp
from jax import lax
from jax.experimental import pallas as pl
from jax.experimental.pallas import tpu as pltpu
from jax.experimental.pallas import tpu_sc as plsc

G = 16
N = 4096
E = 131072
L = 128

NBLK = 8
BN = N // NBLK
EBLK = 128
BE = E // EBLK

_NEG = -1e9


def _leaky(v):
    return jnp.where(v >= 0, v, 0.01 * v)


_NC, _NS = 2, 16
_SC_PARAMS = pltpu.CompilerParams(needs_layout_passes=False)
_NW = _NC * _NS
_EPW = E // _NW
_GCH = 256
_NCH = _EPW // _GCH


def _sc_gather2_body(a_hbm, b_hbm, dst_hbm, src_hbm, adst_hbm, bsrc_hbm,
                     idx0_v, idx1_v, rows0_v, rows1_v, sem0, sem1):
    wid = lax.axis_index("s") * _NC + lax.axis_index("c")
    base0 = wid * _EPW
    idxs = (idx0_v, idx1_v)
    rows = (rows0_v, rows1_v)
    sems = (sem0, sem1)

    def task_refs(j):
        half = j // _NCH
        tab = a_hbm if half == 0 else b_hbm
        ind = dst_hbm if half == 0 else src_hbm
        out = adst_hbm if half == 0 else bsrc_hbm
        off = base0 + (j % _NCH) * _GCH
        return tab, ind, out, off

    tab, ind, out, off = task_refs(0)
    pltpu.sync_copy(ind.at[pl.ds(off, _GCH)], idxs[0])
    pltpu.async_copy(tab.at[idxs[0]], rows[0], sems[0])

    for j in range(2 * _NCH):
        b = j % 2
        nb = (j + 1) % 2
        if j + 1 < 2 * _NCH:
            tab, ind, out, off = task_refs(j + 1)
            pltpu.sync_copy(ind.at[pl.ds(off, _GCH)], idxs[nb])
            pltpu.async_copy(tab.at[idxs[nb]], rows[nb], sems[nb])
        tab, ind, out, off = task_refs(j)
        pltpu.make_async_copy(tab.at[idxs[b]], rows[b], sems[b]).wait()
        pltpu.sync_copy(rows[b], out.at[pl.ds(off, _GCH)])


def _sc_gather2(a, b, dst, src):
    mesh = plsc.VectorSubcoreMesh(core_axis_name="c", subcore_axis_name="s")
    f = pl.kernel(
        _sc_gather2_body,
        mesh=mesh,
        compiler_params=_SC_PARAMS,
        out_type=(
            jax.ShapeDtypeStruct((E, L), jnp.float32),
            jax.ShapeDtypeStruct((E, L), jnp.float32),
        ),
        scratch_types=[
            pltpu.VMEM((_GCH,), jnp.int32),
            pltpu.VMEM((_GCH,), jnp.int32),
            pltpu.VMEM((_GCH, L), jnp.float32),
            pltpu.VMEM((_GCH, L), jnp.float32),
            pltpu.SemaphoreType.DMA,
            pltpu.SemaphoreType.DMA,
        ],
    )
    return f(a, b, dst, src)


CSEG = 256
_FB = 1024
ECAP = E + CSEG
_NPW = N // _NW
_DUMP = _NPW * 8
PREDPAD = 16 * _NW


def _append_flush(buf_refs, hbm_refs, vals, mask, cnt, nf, wbase):
    for br, v in zip(buf_refs, vals):
        plsc.store_compressed(br.at[pl.ds(cnt, 16)], v, mask=mask)
    cnt = cnt + jnp.sum(mask.astype(jnp.int32))

    def flush():
        for br, hr in zip(buf_refs, hbm_refs):
            pltpu.sync_copy(br.at[pl.ds(0, _FB)],
                            hr.at[pl.ds(wbase + nf * _FB, _FB)])
            rem = br[pl.ds(_FB, 16)]
            br[pl.ds(0, 16)] = rem

    jax.lax.cond(cnt >= _FB, flush, lambda: None)
    new_nf = jnp.where(cnt >= _FB, nf + 1, nf)
    new_cnt = jnp.where(cnt >= _FB, cnt - _FB, cnt)
    return new_cnt, new_nf


def _pad_tail(buf_refs, hbm_refs, pads, cnt, nf, wbase):
    base16 = (cnt // 16) * 16
    lanes = lax.iota(jnp.int32, 16)
    for br, padv in zip(buf_refs, pads):
        cur = br[pl.ds(base16, 16)]
        br[pl.ds(base16, 16)] = jnp.where(base16 + lanes < cnt, cur, padv)
        for k in range(1, 18):
            br[pl.ds(base16 + k * 16, 16)] = jnp.zeros((16,), jnp.int32) + padv
    padded = ((cnt + CSEG - 1) // CSEG) * CSEG

    def flush_k(k, _):
        for br, hr in zip(buf_refs, hbm_refs):
            pltpu.sync_copy(br.at[pl.ds(k * CSEG, CSEG)],
                            hr.at[pl.ds(wbase + nf * _FB + k * CSEG, CSEG)])
        return ()

    lax.fori_loop(0, padded // CSEG, flush_k, ())
    return nf * _FB + padded


_PCH = 4096


def _sc_prep_body(dst_hbm, src_hbm,
                  did_hbm, dloc_hbm, dcnt_hbm, addr_hbm, spid_hbm, scnt_hbm,
                  d_v, s_v, did_v, dloc_v, addr_v, spid_v, cnt_v):
    wid = lax.axis_index("s") * _NC + lax.axis_index("c")
    lo = wid * _NPW
    wbase = wid * ECAP

    def chunk(i, carry):
        cnt1, nf1, cnt2, nf2 = carry
        pltpu.sync_copy(dst_hbm.at[pl.ds(i * _PCH, _PCH)], d_v)
        pltpu.sync_copy(src_hbm.at[pl.ds(i * _PCH, _PCH)], s_v)

        def vreg(j, carry2):
            c1, n1, c2, n2 = carry2
            d = d_v[pl.ds(j * 16, 16)]
            s = s_v[pl.ds(j * 16, 16)]
            ids = lax.iota(jnp.int32, 16) + (i * _PCH + j * 16)
            mask_d = (d >= lo) & (d < lo + _NPW)
            c1, n1 = _append_flush((did_v, dloc_v), (did_hbm, dloc_hbm),
                                   (ids, d - lo), mask_d, c1, n1, wbase)
            mask_s = (s >= lo) & (s < lo + _NPW)
            addr = s * N + d
            c2, n2 = _append_flush((addr_v, spid_v), (addr_hbm, spid_hbm),
                                   (addr, ids), mask_s, c2, n2, wbase)
            return c1, n1, c2, n2

        return lax.fori_loop(0, _PCH // 16, vreg, (cnt1, nf1, cnt2, nf2),
                             unroll=2)

    cnt1, nf1, cnt2, nf2 = lax.fori_loop(
        0, E // _PCH, chunk,
        (jnp.int32(0), jnp.int32(0), jnp.int32(0), jnp.int32(0)))

    tot1 = _pad_tail((did_v, dloc_v), (did_hbm, dloc_hbm),
                     (jnp.int32(0), jnp.int32(_DUMP) // 8), cnt1, nf1, wbase)
    tot2 = _pad_tail((addr_v, spid_v), (addr_hbm, spid_hbm),
                     (jnp.int32(N * N) + wid * 16, jnp.int32(0)),
                     cnt2, nf2, wbase)
    cnt_v[...] = jnp.zeros((16,), jnp.int32) + tot1
    pltpu.sync_copy(cnt_v, dcnt_hbm.at[pl.ds(wid * 16, 16)])
    cnt_v[...] = jnp.zeros((16,), jnp.int32) + tot2
    pltpu.sync_copy(cnt_v, scnt_hbm.at[pl.ds(wid * 16, 16)])


def _sc_prep(dst, src):
    mesh = plsc.VectorSubcoreMesh(core_axis_name="c", subcore_axis_name="s")
    lbuf = pltpu.VMEM((_FB + 16 + 288,), jnp.int32)
    f = pl.kernel(
        _sc_prep_body,
        mesh=mesh,
        compiler_params=_SC_PARAMS,
        out_type=(
            jax.ShapeDtypeStruct((_NW * ECAP,), jnp.int32),
            jax.ShapeDtypeStruct((_NW * ECAP,), jnp.int32),
            jax.ShapeDtypeStruct((_NW * 16,), jnp.int32),
            jax.ShapeDtypeStruct((_NW * ECAP,), jnp.int32),
            jax.ShapeDtypeStruct((_NW * ECAP,), jnp.int32),
            jax.ShapeDtypeStruct((_NW * 16,), jnp.int32),
        ),
        scratch_types=[
            pltpu.VMEM((_PCH,), jnp.int32),
            pltpu.VMEM((_PCH,), jnp.int32),
            lbuf, lbuf, lbuf, lbuf,
            pltpu.VMEM((16,), jnp.int32),
        ],
    )
    return f(dst, src)


def _sc_segmax_body(m_hbm, did_hbm, dloc_hbm, dcnt_hbm, aggr_hbm,
                    ids_v, dl_v, rows_v, acc0_v, acc1_v, acc2_v, acc3_v,
                    cnt_v, sem):
    accs = (acc0_v, acc1_v, acc2_v, acc3_v)
    wid = lax.axis_index("s") * _NC + lax.axis_index("c")
    wbase = wid * ECAP

    def initrow(j, _):
        for a in accs:
            for c in range(8):
                a[j, pl.ds(c * 16, 16)] = jnp.full((16,), -jnp.inf, jnp.float32)
        return ()

    lax.fori_loop(0, _NPW + 1, initrow, ())

    pltpu.sync_copy(dcnt_hbm.at[pl.ds(wid * 16, 16)], cnt_v)
    trips = cnt_v[...][0] // CSEG

    def chunk(t, _):
        pltpu.sync_copy(did_hbm.at[pl.ds(wbase + t * CSEG, CSEG)], ids_v)
        pltpu.sync_copy(dloc_hbm.at[pl.ds(wbase + t * CSEG, CSEG)], dl_v)
        pltpu.async_copy(m_hbm.at[ids_v], rows_v, sem).wait()

        def group(g, _):
            dlv = dl_v[pl.ds(g * 16, 16)]
            for k in range(16):
                e = g * 16 + k
                acc = accs[k % 4]
                dl = dlv[k]
                for c in range(8):
                    r = rows_v[e, pl.ds(c * 16, 16)]
                    acc[dl, pl.ds(c * 16, 16)] = jnp.maximum(
                        acc[dl, pl.ds(c * 16, 16)], r)
            return ()

        lax.fori_loop(0, CSEG // 16, group, ())
        return ()

    lax.fori_loop(0, trips, chunk, ())

    def mergerow(j, _):
        for c in range(8):
            m01 = jnp.maximum(acc0_v[j, pl.ds(c * 16, 16)],
                              acc1_v[j, pl.ds(c * 16, 16)])
            m23 = jnp.maximum(acc2_v[j, pl.ds(c * 16, 16)],
                              acc3_v[j, pl.ds(c * 16, 16)])
            acc0_v[j, pl.ds(c * 16, 16)] = jnp.maximum(m01, m23)
        return ()

    lax.fori_loop(0, _NPW, mergerow, ())
    pltpu.sync_copy(acc0_v.at[pl.ds(0, _NPW)],
                    aggr_hbm.at[pl.ds(wid * _NPW, _NPW)])


def _sc_segmax(m, did, dloc, dcnt):
    mesh = plsc.VectorSubcoreMesh(core_axis_name="c", subcore_axis_name="s")
    f = pl.kernel(
        _sc_segmax_body,
        mesh=mesh,
        compiler_params=_SC_PARAMS,
        out_type=jax.ShapeDtypeStruct((N, L), jnp.float32),
        scratch_types=[
            pltpu.VMEM((CSEG,), jnp.int32),
            pltpu.VMEM((CSEG,), jnp.int32),
            pltpu.VMEM((CSEG, L), jnp.float32),
            pltpu.VMEM((_NPW + 1, L), jnp.float32),
            pltpu.VMEM((_NPW + 1, L), jnp.float32),
            pltpu.VMEM((_NPW + 1, L), jnp.float32),
            pltpu.VMEM((_NPW + 1, L), jnp.float32),
            pltpu.VMEM((16,), jnp.int32),
            pltpu.SemaphoreType.DMA,
        ],
    )
    return f(m, did, dloc, dcnt)


_FILLB = 16384
_RPW = N * N // _NW


def _sc_pred_body(pout_hbm, addr_hbm, spid_hbm, scnt_hbm, pred_hbm,
                  fill_v, addr_v, pid_v, vals_v, cnt_v, sem):
    wid = lax.axis_index("s") * _NC + lax.axis_index("c")
    wbase = wid * ECAP

    def initf(j, _):
        fill_v[pl.ds(j * 16, 16)] = jnp.full((16,), _NEG, jnp.float32)
        return ()

    lax.fori_loop(0, _FILLB // 16, initf, ())

    def fill(k, _):
        pltpu.async_copy(fill_v,
                         pred_hbm.at[pl.ds(wid * _RPW + k * _FILLB, _FILLB)],
                         sem)
        return ()

    lax.fori_loop(0, _RPW // _FILLB, fill, ())

    def fill_wait(k, _):
        pltpu.make_async_copy(
            fill_v, pred_hbm.at[pl.ds(wid * _RPW + k * _FILLB, _FILLB)],
            sem).wait()
        return ()

    lax.fori_loop(0, _RPW // _FILLB, fill_wait, ())

    pltpu.sync_copy(scnt_hbm.at[pl.ds(wid * 16, 16)], cnt_v)
    trips = cnt_v[...][0] // CSEG

    def chunk(t, _):
        pltpu.sync_copy(addr_hbm.at[pl.ds(wbase + t * CSEG, CSEG)], addr_v)
        pltpu.sync_copy(spid_hbm.at[pl.ds(wbase + t * CSEG, CSEG)], pid_v)
        pltpu.async_copy(pout_hbm.at[pid_v], vals_v, sem).wait()
        pltpu.async_copy(vals_v, pred_hbm.at[addr_v], sem).wait()
        return ()

    lax.fori_loop(0, trips, chunk, ())


def _sc_pred(pout, addr, spid, scnt):
    mesh = plsc.VectorSubcoreMesh(core_axis_name="c", subcore_axis_name="s")
    f = pl.kernel(
        _sc_pred_body,
        mesh=mesh,
        compiler_params=_SC_PARAMS,
        out_type=jax.ShapeDtypeStruct((N * N + PREDPAD,), jnp.float32),
        scratch_types=[
            pltpu.VMEM((_FILLB,), jnp.float32),
            pltpu.VMEM((CSEG,), jnp.int32),
            pltpu.VMEM((CSEG,), jnp.int32),
            pltpu.VMEM((CSEG,), jnp.float32),
            pltpu.VMEM((16,), jnp.int32),
            pltpu.SemaphoreType.DMA,
        ],
    )
    return f(pout, addr, spid, scnt)


def _prep_body(pt_ref, h_ref, w0_ref, wenc1_ref, wm1d_ref, wm1s_ref,
               enc_ref, a_ref, b_ref):
    enc = jax.nn.relu(pt_ref[...] * w0_ref[...] +
                      jnp.dot(h_ref[...], wenc1_ref[...],
                              preferred_element_type=jnp.float32))
    enc_ref[...] = enc
    a_ref[...] = jnp.dot(enc, wm1d_ref[...], preferred_element_type=jnp.float32)
    b_ref[...] = jnp.dot(enc, wm1s_ref[...], preferred_element_type=jnp.float32)


def _prep(pt, h, w0, wenc1, wm1d, wm1s):
    row = pl.BlockSpec((BN, L), lambda i: (i, 0))
    col = pl.BlockSpec((BN, 1), lambda i: (i, 0))
    full = pl.BlockSpec((L, L), lambda i: (0, 0))
    vec = pl.BlockSpec((1, L), lambda i: (0, 0))
    return pl.pallas_call(
        _prep_body,
        grid=(NBLK,),
        in_specs=[col, row, vec, full, full, full],
        out_specs=[row, row, row],
        out_shape=[jax.ShapeDtypeStruct((N, L), jnp.float32)] * 3,
    )(pt, h, w0, wenc1, wm1d, wm1s)


def _mmlp_body(a_ref, b_ref, ea_ref, w_ref, wm2_ref, o_ref):
    pre = a_ref[...] + b_ref[...] + ea_ref[...] * w_ref[...]
    q = _leaky(pre)
    o_ref[...] = _leaky(jnp.dot(q, wm2_ref[...],
                                preferred_element_type=jnp.float32))


def _mmlp(adst, bsrc, ea2, w, wm2):
    row = pl.BlockSpec((BE, L), lambda i: (i, 0))
    col = pl.BlockSpec((BE, 1), lambda i: (i, 0))
    full = pl.BlockSpec((L, L), lambda i: (0, 0))
    vec = pl.BlockSpec((1, L), lambda i: (0, 0))
    return pl.pallas_call(
        _mmlp_body,
        grid=(EBLK,),
        in_specs=[row, row, col, vec, full],
        out_specs=row,
        out_shape=jax.ShapeDtypeStruct((E, L), jnp.float32),
    )(adst, bsrc, ea2, w, wm2)


def _gru(u, gh_ref, h, w_iht_ref):
    gi = jnp.dot(u, w_iht_ref[...], preferred_element_type=jnp.float32)
    gh = gh_ref
    r = jax.nn.sigmoid(gi[:, :L] + gh[:, :L])
    z = jax.nn.sigmoid(gi[:, L:2 * L] + gh[:, L:2 * L])
    ng = jnp.tanh(gi[:, 2 * L:] + r * gh[:, 2 * L:])
    return (1.0 - z) * ng + z * h


def _update_body(enc_ref, raw_ref, h_ref, wue_ref, wua_ref, wiht_ref,
                 whht_ref, wmste_ref, wmsth_ref, hn_ref, mst_ref):
    raw = raw_ref[...]
    aggr = jnp.where(raw > -jnp.inf, raw, 0.0)
    enc = enc_ref[...]
    h = h_ref[...]
    u = _leaky(jnp.dot(enc, wue_ref[...], preferred_element_type=jnp.float32) +
               jnp.dot(aggr, wua_ref[...], preferred_element_type=jnp.float32))
    gh = jnp.dot(h, whht_ref[...], preferred_element_type=jnp.float32)
    hn = _gru(u, gh, h, wiht_ref)
    hn_ref[...] = hn
    mst_ref[...] = (jnp.dot(enc, wmste_ref[...], preferred_element_type=jnp.float32) +
                    jnp.dot(hn, wmsth_ref[...], preferred_element_type=jnp.float32))


def _update(enc, raw, h, wue, wua, wiht, whht, wmste, wmsth):
    row = pl.BlockSpec((BN, L), lambda i: (i, 0))
    full = pl.BlockSpec((L, L), lambda i: (0, 0))
    full3 = pl.BlockSpec((L, 3 * L), lambda i: (0, 0))
    cvec = pl.BlockSpec((L, 1), lambda i: (0, 0))
    col = pl.BlockSpec((BN, 1), lambda i: (i, 0))
    return pl.pallas_call(
        _update_body,
        grid=(NBLK,),
        in_specs=[row, row, row, full, full, full3, full3, cvec, cvec],
        out_specs=[row, col],
        out_shape=[jax.ShapeDtypeStruct((N, L), jnp.float32),
                   jax.ShapeDtypeStruct((N, 1), jnp.float32)],
    )(enc, raw, h, wue, wua, wiht, whht, wmste, wmsth)


def _update0_body(emax_ref, emin_ref, wm1e_ref, wm2_ref, wua_ref, wiht_ref,
                  wmsth_ref, hn_ref, mst_ref):
    g = _leaky(jnp.dot(_leaky(wm1e_ref[...]), wm2_ref[...],
                       preferred_element_type=jnp.float32))
    emax_raw = emax_ref[...]
    emin_raw = emin_ref[...]
    emax = jnp.where(emax_raw > -jnp.inf, emax_raw, 0.0)
    emin = jnp.where(emin_raw < jnp.inf, emin_raw, 0.0)
    aggr = jnp.where(g > 0, emax * g, emin * g)
    u = _leaky(jnp.dot(aggr, wua_ref[...], preferred_element_type=jnp.float32))
    gi = jnp.dot(u, wiht_ref[...], preferred_element_type=jnp.float32)
    z = jax.nn.sigmoid(gi[:, L:2 * L])
    ng = jnp.tanh(gi[:, 2 * L:])
    hn = (1.0 - z) * ng
    hn_ref[...] = hn
    mst_ref[...] = jnp.dot(hn, wmsth_ref[...], preferred_element_type=jnp.float32)


def _update0(emax, emin, wm1e, wm2, wua, wiht, wmsth):
    row = pl.BlockSpec((BN, L), lambda i: (i, 0))
    col = pl.BlockSpec((BN, 1), lambda i: (i, 0))
    full = pl.BlockSpec((L, L), lambda i: (0, 0))
    full3 = pl.BlockSpec((L, 3 * L), lambda i: (0, 0))
    vec = pl.BlockSpec((1, L), lambda i: (0, 0))
    cvec = pl.BlockSpec((L, 1), lambda i: (0, 0))
    return pl.pallas_call(
        _update0_body,
        grid=(NBLK,),
        in_specs=[col, col, vec, full, full, full3, cvec],
        out_specs=[row, col],
        out_shape=[jax.ShapeDtypeStruct((N, L), jnp.float32),
                   jax.ShapeDtypeStruct((N, 1), jnp.float32)],
    )(emax, emin, wm1e, wm2, wua, wiht, wmsth)


def _select_body(mst_ref, pt_ref, out_ref):
    mst = mst_ref[...]
    pt = pt_ref[...]
    nt = jnp.where(pt != 0, _NEG, mst)
    iota = lax.broadcasted_iota(jnp.int32, (G, N // G), 1)
    rowmax = jnp.max(nt, axis=1, keepdims=True)
    cand = jnp.where(nt == rowmax, iota, jnp.int32(2**30))
    chosen = jnp.min(cand, axis=1, keepdims=True)
    out_ref[...] = jnp.where(iota == chosen, 1.0, pt)


def _select(mst_g, pt_g):
    blk = pl.BlockSpec((G, N // G), lambda: (0, 0))
    return pl.pallas_call(
        _select_body,
        in_specs=[blk, blk],
        out_specs=blk,
        out_shape=jax.ShapeDtypeStruct((G, N // G), jnp.float32),
    )(mst_g, pt_g)


def _final_body(enc_ref, raw_ref, h_ref, wue_ref, wua_ref, wiht_ref,
                whht_ref, wp1s_ref, wp1d_ref, c_ref, d_ref):
    raw = raw_ref[...]
    aggr = jnp.where(raw > -jnp.inf, raw, 0.0)
    enc = enc_ref[...]
    h = h_ref[...]
    u = _leaky(jnp.dot(enc, wue_ref[...], preferred_element_type=jnp.float32) +
               jnp.dot(aggr, wua_ref[...], preferred_element_type=jnp.float32))
    gh = jnp.dot(h, whht_ref[...], preferred_element_type=jnp.float32)
    hn = _gru(u, gh, h, wiht_ref)
    c_ref[...] = jnp.dot(hn, wp1s_ref[...], preferred_element_type=jnp.float32)
    d_ref[...] = jnp.dot(hn, wp1d_ref[...], preferred_element_type=jnp.float32)


def _final_update(enc, raw, h, wue, wua, wiht, whht, wp1s, wp1d):
    row = pl.BlockSpec((BN, L), lambda i: (i, 0))
    full = pl.BlockSpec((L, L), lambda i: (0, 0))
    full3 = pl.BlockSpec((L, 3 * L), lambda i: (0, 0))
    return pl.pallas_call(
        _final_body,
        grid=(NBLK,),
        in_specs=[row, row, row, full, full, full3, full3, full, full],
        out_specs=[row, row],
        out_shape=[jax.ShapeDtypeStruct((N, L), jnp.float32)] * 2,
    )(enc, raw, h, wue, wua, wiht, whht, wp1s, wp1d)


def _pout_body(c_ref, d_ref, ea_ref, w_ref, wp2_ref, o_ref):
    pin = c_ref[...] + d_ref[...] + ea_ref[...] * w_ref[...]
    o_ref[...] = jnp.dot(jax.nn.relu(pin), wp2_ref[...],
                         preferred_element_type=jnp.float32)


def _pout(csrc, ddst, ea2, wp1e, wp2):
    row = pl.BlockSpec((BE, L), lambda i: (i, 0))
    col = pl.BlockSpec((BE, 1), lambda i: (i, 0))
    vec = pl.BlockSpec((1, L), lambda i: (0, 0))
    cvec = pl.BlockSpec((L, 1), lambda i: (0, 0))
    return pl.pallas_call(
        _pout_body,
        grid=(EBLK,),
        in_specs=[row, row, col, vec, cvec],
        out_specs=col,
        out_shape=jax.ShapeDtypeStruct((E, 1), jnp.float32),
    )(csrc, ddst, ea2, wp1e, wp2)


def kernel(x, edge_attr, edge_index, W_enc, W_m1, W_m2, W_u, W_ih, W_hh, W_mst, W_p1, W_p2):
    n = x.shape[0]
    steps = x.shape[1]
    src = edge_index[0]
    dst = edge_index[1]
    ea = edge_attr
    ea2 = ea[:, None]

    w_enc0 = W_enc[0:1]
    W_enc1 = W_enc[1:]
    Wm1_d = W_m1[:L]
    Wm1_s = W_m1[L:2 * L]
    wm1_e = W_m1[2 * L:2 * L + 1]
    Wu_e = W_u[:L]
    Wu_a = W_u[L:]
    W_ihT = W_ih.T
    W_hhT = W_hh.T
    Wmst_e = W_mst[:L]
    Wmst_h = W_mst[L:]
    Wp1_s = W_p1[:L]
    Wp1_d = W_p1[L:2 * L]
    wp1_e = W_p1[2 * L:2 * L + 1]

    pt = x[:, 0]

    did, dloc, dcnt, addr, spid, scnt = _sc_prep(dst, src)

    ea_max = jax.ops.segment_max(ea, dst, num_segments=n)[:, None]
    ea_min = jax.ops.segment_min(ea, dst, num_segments=n)[:, None]
    h, mst = _update0(ea_max, ea_min, wm1_e, W_m2, Wu_a, W_ihT, Wmst_h)
    pt = _select(mst.reshape(G, n // G), pt.reshape(G, n // G)).reshape(-1)

    for step in range(1, steps):
        enc, A, B = _prep(pt[:, None], h, w_enc0, W_enc1, Wm1_d, Wm1_s)
        adst, bsrc = _sc_gather2(A, B, dst, src)
        m = _mmlp(adst, bsrc, ea2, wm1_e, W_m2)
        raw = _sc_segmax(m, did, dloc, dcnt)
        if step < steps - 1:
            h, mst = _update(enc, raw, h, Wu_e, Wu_a, W_ihT, W_hhT,
                             Wmst_e, Wmst_h)
            pt = _select(mst.reshape(G, n // G),
                         pt.reshape(G, n // G)).reshape(-1)
        else:
            C, D = _final_update(enc, raw, h, Wu_e, Wu_a, W_ihT, W_hhT,
                                 Wp1_s, Wp1_d)

    csrc, ddst = _sc_gather2(C, D, src, dst)
    p_out = _pout(csrc, ddst, ea2, wp1_e, W_p2)[:, 0]
    pred_flat = _sc_pred(p_out, addr, spid, scnt)
    return pred_flat[:n * n].reshape(n, n)


# speedup vs baseline: 1.0725x; 1.0616x over previous
import functools

import jax
import jax.numpy as jnp
from jax import lax
from jax.experimental import pallas as pl
from jax.experimental.pallas import tpu as pltpu
from jax.experimental.pallas import tpu_sc as plsc

G = 16
N = 4096
E = 131072
L = 128

NBLK = 8
BN = N // NBLK
EBLK = 128
BE = E // EBLK

_NEG = -1e9


def _leaky(v):
    return jnp.where(v >= 0, v, 0.01 * v)


_NC, _NS = 2, 16
_SC_PARAMS = pltpu.CompilerParams(needs_layout_passes=False)
_NW = _NC * _NS
_EPW = E // _NW
_GCH = 256
_NCH = _EPW // _GCH


def _sc_gather2_body(a_hbm, b_hbm, dst_hbm, src_hbm, adst_hbm, bsrc_hbm,
                     idx0_v, idx1_v, rows0_v, rows1_v, sem0, sem1):
    wid = lax.axis_index("s") * _NC + lax.axis_index("c")
    base0 = wid * _EPW
    idxs = (idx0_v, idx1_v)
    rows = (rows0_v, rows1_v)
    sems = (sem0, sem1)

    def task_refs(j):
        half = j // _NCH
        tab = a_hbm if half == 0 else b_hbm
        ind = dst_hbm if half == 0 else src_hbm
        out = adst_hbm if half == 0 else bsrc_hbm
        off = base0 + (j % _NCH) * _GCH
        return tab, ind, out, off

    tab, ind, out, off = task_refs(0)
    pltpu.sync_copy(ind.at[pl.ds(off, _GCH)], idxs[0])
    pltpu.async_copy(tab.at[idxs[0]], rows[0], sems[0])

    for j in range(2 * _NCH):
        b = j % 2
        nb = (j + 1) % 2
        if j + 1 < 2 * _NCH:
            tab, ind, out, off = task_refs(j + 1)
            pltpu.sync_copy(ind.at[pl.ds(off, _GCH)], idxs[nb])
            pltpu.async_copy(tab.at[idxs[nb]], rows[nb], sems[nb])
        tab, ind, out, off = task_refs(j)
        pltpu.make_async_copy(tab.at[idxs[b]], rows[b], sems[b]).wait()
        pltpu.sync_copy(rows[b], out.at[pl.ds(off, _GCH)])


def _sc_gather2(a, b, dst, src):
    mesh = plsc.VectorSubcoreMesh(core_axis_name="c", subcore_axis_name="s")
    f = pl.kernel(
        _sc_gather2_body,
        mesh=mesh,
        compiler_params=_SC_PARAMS,
        out_type=(
            jax.ShapeDtypeStruct((E, L), jnp.float32),
            jax.ShapeDtypeStruct((E, L), jnp.float32),
        ),
        scratch_types=[
            pltpu.VMEM((_GCH,), jnp.int32),
            pltpu.VMEM((_GCH,), jnp.int32),
            pltpu.VMEM((_GCH, L), jnp.float32),
            pltpu.VMEM((_GCH, L), jnp.float32),
            pltpu.SemaphoreType.DMA,
            pltpu.SemaphoreType.DMA,
        ],
    )
    return f(a, b, dst, src)


CSEG = 256
_FB = 1024
ECAP = E + CSEG
_NPW = N // _NW
_DUMP = _NPW * 8
PREDPAD = 16 * _NW


def _append_flush(buf_refs, hbm_refs, vals, mask, cnt, nf, wbase):
    for br, v in zip(buf_refs, vals):
        plsc.store_compressed(br.at[pl.ds(cnt, 16)], v, mask=mask)
    cnt = cnt + jnp.sum(mask.astype(jnp.int32))

    def flush():
        for br, hr in zip(buf_refs, hbm_refs):
            pltpu.sync_copy(br.at[pl.ds(0, _FB)],
                            hr.at[pl.ds(wbase + nf * _FB, _FB)])
            rem = br[pl.ds(_FB, 16)]
            br[pl.ds(0, 16)] = rem

    jax.lax.cond(cnt >= _FB, flush, lambda: None)
    new_nf = jnp.where(cnt >= _FB, nf + 1, nf)
    new_cnt = jnp.where(cnt >= _FB, cnt - _FB, cnt)
    return new_cnt, new_nf


def _pad_tail(buf_refs, hbm_refs, pads, cnt, nf, wbase):
    base16 = (cnt // 16) * 16
    lanes = lax.iota(jnp.int32, 16)
    for br, padv in zip(buf_refs, pads):
        cur = br[pl.ds(base16, 16)]
        br[pl.ds(base16, 16)] = jnp.where(base16 + lanes < cnt, cur, padv)
        for k in range(1, 18):
            br[pl.ds(base16 + k * 16, 16)] = jnp.zeros((16,), jnp.int32) + padv
    padded = ((cnt + CSEG - 1) // CSEG) * CSEG

    def flush_k(k, _):
        for br, hr in zip(buf_refs, hbm_refs):
            pltpu.sync_copy(br.at[pl.ds(k * CSEG, CSEG)],
                            hr.at[pl.ds(wbase + nf * _FB + k * CSEG, CSEG)])
        return ()

    lax.fori_loop(0, padded // CSEG, flush_k, ())
    return nf * _FB + padded


_PCH = 4096


def _sc_prep_body(dst_hbm, src_hbm,
                  did_hbm, dcnt_hbm, addr_hbm, spid_hbm, scnt_hbm,
                  d_v, s_v, did_v, addr_v, spid_v, cnt_v):
    wid = lax.axis_index("s") * _NC + lax.axis_index("c")
    lo = wid * _NPW
    wbase = wid * ECAP

    def chunk(i, carry):
        cnt1, nf1, cnt2, nf2 = carry
        pltpu.sync_copy(dst_hbm.at[pl.ds(i * _PCH, _PCH)], d_v)
        pltpu.sync_copy(src_hbm.at[pl.ds(i * _PCH, _PCH)], s_v)

        def vreg(j, carry2):
            c1, n1, c2, n2 = carry2
            d = d_v[pl.ds(j * 16, 16)]
            s = s_v[pl.ds(j * 16, 16)]
            ids = lax.iota(jnp.int32, 16) + (i * _PCH + j * 16)
            mask_d = (d >= lo) & (d < lo + _NPW)
            packed = ids | ((d - lo) << 18)
            c1, n1 = _append_flush((did_v,), (did_hbm,),
                                   (packed,), mask_d, c1, n1, wbase)
            mask_s = (s >= lo) & (s < lo + _NPW)
            addr = s * N + d
            c2, n2 = _append_flush((addr_v, spid_v), (addr_hbm, spid_hbm),
                                   (addr, ids), mask_s, c2, n2, wbase)
            return c1, n1, c2, n2

        return lax.fori_loop(0, _PCH // 16, vreg, (cnt1, nf1, cnt2, nf2),
                             unroll=2)

    cnt1, nf1, cnt2, nf2 = lax.fori_loop(
        0, E // _PCH, chunk,
        (jnp.int32(0), jnp.int32(0), jnp.int32(0), jnp.int32(0)))

    tot1 = _pad_tail((did_v,), (did_hbm,),
                     (jnp.int32(_NPW << 18),), cnt1, nf1, wbase)
    tot2 = _pad_tail((addr_v, spid_v), (addr_hbm, spid_hbm),
                     (jnp.int32(N * N) + wid * 16, jnp.int32(0)),
                     cnt2, nf2, wbase)
    cnt_v[...] = jnp.zeros((16,), jnp.int32) + tot1
    pltpu.sync_copy(cnt_v, dcnt_hbm.at[pl.ds(wid * 16, 16)])
    cnt_v[...] = jnp.zeros((16,), jnp.int32) + tot2
    pltpu.sync_copy(cnt_v, scnt_hbm.at[pl.ds(wid * 16, 16)])


def _sc_prep(dst, src):
    mesh = plsc.VectorSubcoreMesh(core_axis_name="c", subcore_axis_name="s")
    lbuf = pltpu.VMEM((_FB + 16 + 288,), jnp.int32)
    f = pl.kernel(
        _sc_prep_body,
        mesh=mesh,
        compiler_params=_SC_PARAMS,
        out_type=(
            jax.ShapeDtypeStruct((_NW * ECAP,), jnp.int32),
            jax.ShapeDtypeStruct((_NW * 16,), jnp.int32),
            jax.ShapeDtypeStruct((_NW * ECAP,), jnp.int32),
            jax.ShapeDtypeStruct((_NW * ECAP,), jnp.int32),
            jax.ShapeDtypeStruct((_NW * 16,), jnp.int32),
        ),
        scratch_types=[
            pltpu.VMEM((_PCH,), jnp.int32),
            pltpu.VMEM((_PCH,), jnp.int32),
            lbuf, lbuf, lbuf,
            pltpu.VMEM((16,), jnp.int32),
        ],
    )
    return f(dst, src)


def _sc_segmax_body(m_hbm, did_hbm, dcnt_hbm, aggr_hbm,
                    pk0_v, pk1_v, ids0_v, ids1_v, rows0_v, rows1_v,
                    acc0_v, acc1_v,
                    cnt_v, lsem0, lsem1, gsem0, gsem1):
    accs = (acc0_v, acc1_v)
    pks = (pk0_v, pk1_v)
    idss = (ids0_v, ids1_v)
    rows = (rows0_v, rows1_v)
    lsems = (lsem0, lsem1)
    gsems = (gsem0, gsem1)
    wid = lax.axis_index("s") * _NC + lax.axis_index("c")
    wbase = wid * ECAP

    def initrow(j, _):
        for a in accs:
            for c in range(8):
                a[j, pl.ds(c * 16, 16)] = jnp.full((16,), -jnp.inf, jnp.float32)
        return ()

    lax.fori_loop(0, _NPW + 1, initrow, ())

    pltpu.sync_copy(dcnt_hbm.at[pl.ds(wid * 16, 16)], cnt_v)
    trips = cnt_v[...][0] // CSEG

    def stage(t, b):
        pltpu.make_async_copy(did_hbm.at[pl.ds(wbase + t * CSEG, CSEG)],
                              pks[b], lsems[b]).wait()
        for g in range(CSEG // 16):
            p = pks[b][pl.ds(g * 16, 16)]
            idss[b][pl.ds(g * 16, 16)] = p & ((1 << 18) - 1)
        pltpu.async_copy(m_hbm.at[idss[b]], rows[b], gsems[b])

    def compute(b):
        def group(g, _):
            dlv = pks[b][pl.ds(g * 16, 16)] >> 18
            for k in range(16):
                e = g * 16 + k
                acc = accs[k % 2]
                dl = dlv[k]
                for c in range(8):
                    r = rows[b][e, pl.ds(c * 16, 16)]
                    acc[dl, pl.ds(c * 16, 16)] = jnp.maximum(
                        acc[dl, pl.ds(c * 16, 16)], r)
            return ()

        lax.fori_loop(0, CSEG // 16, group, ())

    @pl.when(trips > 0)
    def _():
        pltpu.async_copy(did_hbm.at[pl.ds(wbase, CSEG)], pks[0], lsems[0])
        stage(0, 0)

    def chunk(t, _):
        def do(b, nb):
            @pl.when(t + 1 < trips)
            def _():
                pltpu.async_copy(
                    did_hbm.at[pl.ds(wbase + (t + 1) * CSEG, CSEG)],
                    pks[nb], lsems[nb])
                stage(t + 1, nb)
            pltpu.make_async_copy(m_hbm.at[idss[b]], rows[b], gsems[b]).wait()
            compute(b)

        @pl.when(t % 2 == 0)
        def _():
            do(0, 1)

        @pl.when(t % 2 == 1)
        def _():
            do(1, 0)

        return ()

    lax.fori_loop(0, trips, chunk, ())

    def mergerow(j, _):
        for c in range(8):
            acc0_v[j, pl.ds(c * 16, 16)] = jnp.maximum(
                acc0_v[j, pl.ds(c * 16, 16)],
                acc1_v[j, pl.ds(c * 16, 16)])
        return ()

    lax.fori_loop(0, _NPW, mergerow, ())
    pltpu.sync_copy(acc0_v.at[pl.ds(0, _NPW)],
                    aggr_hbm.at[pl.ds(wid * _NPW, _NPW)])


def _sc_segmax(m, did, dcnt):
    mesh = plsc.VectorSubcoreMesh(core_axis_name="c", subcore_axis_name="s")
    f = pl.kernel(
        _sc_segmax_body,
        mesh=mesh,
        compiler_params=_SC_PARAMS,
        out_type=jax.ShapeDtypeStruct((N, L), jnp.float32),
        scratch_types=[
            pltpu.VMEM((CSEG,), jnp.int32),
            pltpu.VMEM((CSEG,), jnp.int32),
            pltpu.VMEM((CSEG,), jnp.int32),
            pltpu.VMEM((CSEG,), jnp.int32),
            pltpu.VMEM((CSEG, L), jnp.float32),
            pltpu.VMEM((CSEG, L), jnp.float32),
            pltpu.VMEM((_NPW + 1, L), jnp.float32),
            pltpu.VMEM((_NPW + 1, L), jnp.float32),
            pltpu.VMEM((16,), jnp.int32),
            pltpu.SemaphoreType.DMA,
            pltpu.SemaphoreType.DMA,
            pltpu.SemaphoreType.DMA,
            pltpu.SemaphoreType.DMA,
        ],
    )
    return f(m, did, dcnt)


_FILLB = 16384
_RPW = N * N // _NW


def _sc_pred_body(pout_hbm, addr_hbm, spid_hbm, scnt_hbm, pred_hbm,
                  fill_v, addr_v, pid_v, vals_v, cnt_v, sem):
    wid = lax.axis_index("s") * _NC + lax.axis_index("c")
    wbase = wid * ECAP

    def initf(j, _):
        fill_v[pl.ds(j * 16, 16)] = jnp.full((16,), _NEG, jnp.float32)
        return ()

    lax.fori_loop(0, _FILLB // 16, initf, ())

    def fill(k, _):
        pltpu.async_copy(fill_v,
                         pred_hbm.at[pl.ds(wid * _RPW + k * _FILLB, _FILLB)],
                         sem)
        return ()

    lax.fori_loop(0, _RPW // _FILLB, fill, ())

    def fill_wait(k, _):
        pltpu.make_async_copy(
            fill_v, pred_hbm.at[pl.ds(wid * _RPW + k * _FILLB, _FILLB)],
            sem).wait()
        return ()

    lax.fori_loop(0, _RPW // _FILLB, fill_wait, ())

    pltpu.sync_copy(scnt_hbm.at[pl.ds(wid * 16, 16)], cnt_v)
    trips = cnt_v[...][0] // CSEG

    def chunk(t, _):
        pltpu.sync_copy(addr_hbm.at[pl.ds(wbase + t * CSEG, CSEG)], addr_v)
        pltpu.sync_copy(spid_hbm.at[pl.ds(wbase + t * CSEG, CSEG)], pid_v)
        pltpu.async_copy(pout_hbm.at[pid_v], vals_v, sem).wait()
        pltpu.async_copy(vals_v, pred_hbm.at[addr_v], sem).wait()
        return ()

    lax.fori_loop(0, trips, chunk, ())


def _sc_pred(pout, addr, spid, scnt):
    mesh = plsc.VectorSubcoreMesh(core_axis_name="c", subcore_axis_name="s")
    f = pl.kernel(
        _sc_pred_body,
        mesh=mesh,
        compiler_params=_SC_PARAMS,
        out_type=jax.ShapeDtypeStruct((N * N + PREDPAD,), jnp.float32),
        scratch_types=[
            pltpu.VMEM((_FILLB,), jnp.float32),
            pltpu.VMEM((CSEG,), jnp.int32),
            pltpu.VMEM((CSEG,), jnp.int32),
            pltpu.VMEM((CSEG,), jnp.float32),
            pltpu.VMEM((16,), jnp.int32),
            pltpu.SemaphoreType.DMA,
        ],
    )
    return f(pout, addr, spid, scnt)


def _prep_body(pt_ref, h_ref, w0_ref, wenc1_ref, wm1d_ref, wm1s_ref,
               enc_ref, a_ref, b_ref):
    enc = jax.nn.relu(pt_ref[...] * w0_ref[...] +
                      jnp.dot(h_ref[...], wenc1_ref[...],
                              preferred_element_type=jnp.float32))
    enc_ref[...] = enc
    a_ref[...] = jnp.dot(enc, wm1d_ref[...], preferred_element_type=jnp.float32)
    b_ref[...] = jnp.dot(enc, wm1s_ref[...], preferred_element_type=jnp.float32)


def _prep(pt, h, w0, wenc1, wm1d, wm1s):
    row = pl.BlockSpec((BN, L), lambda i: (i, 0))
    col = pl.BlockSpec((BN, 1), lambda i: (i, 0))
    full = pl.BlockSpec((L, L), lambda i: (0, 0))
    vec = pl.BlockSpec((1, L), lambda i: (0, 0))
    return pl.pallas_call(
        _prep_body,
        grid=(NBLK,),
        in_specs=[col, row, vec, full, full, full],
        out_specs=[row, row, row],
        out_shape=[jax.ShapeDtypeStruct((N, L), jnp.float32)] * 3,
    )(pt, h, w0, wenc1, wm1d, wm1s)


def _mmlp_body(a_ref, b_ref, ea_ref, w_ref, wm2_ref, o_ref):
    pre = a_ref[...] + b_ref[...] + ea_ref[...] * w_ref[...]
    q = _leaky(pre)
    o_ref[...] = _leaky(jnp.dot(q, wm2_ref[...],
                                preferred_element_type=jnp.float32))


def _mmlp(adst, bsrc, ea2, w, wm2):
    row = pl.BlockSpec((BE, L), lambda i: (i, 0))
    col = pl.BlockSpec((BE, 1), lambda i: (i, 0))
    full = pl.BlockSpec((L, L), lambda i: (0, 0))
    vec = pl.BlockSpec((1, L), lambda i: (0, 0))
    return pl.pallas_call(
        _mmlp_body,
        grid=(EBLK,),
        in_specs=[row, row, col, vec, full],
        out_specs=row,
        out_shape=jax.ShapeDtypeStruct((E, L), jnp.float32),
    )(adst, bsrc, ea2, w, wm2)


def _gru(u, gh_ref, h, w_iht_ref):
    gi = jnp.dot(u, w_iht_ref[...], preferred_element_type=jnp.float32)
    gh = gh_ref
    r = jax.nn.sigmoid(gi[:, :L] + gh[:, :L])
    z = jax.nn.sigmoid(gi[:, L:2 * L] + gh[:, L:2 * L])
    ng = jnp.tanh(gi[:, 2 * L:] + r * gh[:, 2 * L:])
    return (1.0 - z) * ng + z * h


def _update_body(enc_ref, raw_ref, h_ref, wue_ref, wua_ref, wiht_ref,
                 whht_ref, wmste_ref, wmsth_ref, hn_ref, mst_ref):
    raw = raw_ref[...]
    aggr = jnp.where(raw > -jnp.inf, raw, 0.0)
    enc = enc_ref[...]
    h = h_ref[...]
    u = _leaky(jnp.dot(enc, wue_ref[...], preferred_element_type=jnp.float32) +
               jnp.dot(aggr, wua_ref[...], preferred_element_type=jnp.float32))
    gh = jnp.dot(h, whht_ref[...], preferred_element_type=jnp.float32)
    hn = _gru(u, gh, h, wiht_ref)
    hn_ref[...] = hn
    mst_ref[...] = (jnp.dot(enc, wmste_ref[...], preferred_element_type=jnp.float32) +
                    jnp.dot(hn, wmsth_ref[...], preferred_element_type=jnp.float32))


def _update(enc, raw, h, wue, wua, wiht, whht, wmste, wmsth):
    row = pl.BlockSpec((BN, L), lambda i: (i, 0))
    full = pl.BlockSpec((L, L), lambda i: (0, 0))
    full3 = pl.BlockSpec((L, 3 * L), lambda i: (0, 0))
    cvec = pl.BlockSpec((L, 1), lambda i: (0, 0))
    col = pl.BlockSpec((BN, 1), lambda i: (i, 0))
    return pl.pallas_call(
        _update_body,
        grid=(NBLK,),
        in_specs=[row, row, row, full, full, full3, full3, cvec, cvec],
        out_specs=[row, col],
        out_shape=[jax.ShapeDtypeStruct((N, L), jnp.float32),
                   jax.ShapeDtypeStruct((N, 1), jnp.float32)],
    )(enc, raw, h, wue, wua, wiht, whht, wmste, wmsth)


def _update0_body(emax_ref, emin_ref, wm1e_ref, wm2_ref, wua_ref, wiht_ref,
                  wmsth_ref, hn_ref, mst_ref):
    g = _leaky(jnp.dot(_leaky(wm1e_ref[...]), wm2_ref[...],
                       preferred_element_type=jnp.float32))
    emax_raw = emax_ref[...]
    emin_raw = emin_ref[...]
    emax = jnp.where(emax_raw > -jnp.inf, emax_raw, 0.0)
    emin = jnp.where(emin_raw < jnp.inf, emin_raw, 0.0)
    aggr = jnp.where(g > 0, emax * g, emin * g)
    u = _leaky(jnp.dot(aggr, wua_ref[...], preferred_element_type=jnp.float32))
    gi = jnp.dot(u, wiht_ref[...], preferred_element_type=jnp.float32)
    z = jax.nn.sigmoid(gi[:, L:2 * L])
    ng = jnp.tanh(gi[:, 2 * L:])
    hn = (1.0 - z) * ng
    hn_ref[...] = hn
    mst_ref[...] = jnp.dot(hn, wmsth_ref[...], preferred_element_type=jnp.float32)


def _update0(emax, emin, wm1e, wm2, wua, wiht, wmsth):
    row = pl.BlockSpec((BN, L), lambda i: (i, 0))
    col = pl.BlockSpec((BN, 1), lambda i: (i, 0))
    full = pl.BlockSpec((L, L), lambda i: (0, 0))
    full3 = pl.BlockSpec((L, 3 * L), lambda i: (0, 0))
    vec = pl.BlockSpec((1, L), lambda i: (0, 0))
    cvec = pl.BlockSpec((L, 1), lambda i: (0, 0))
    return pl.pallas_call(
        _update0_body,
        grid=(NBLK,),
        in_specs=[col, col, vec, full, full, full3, cvec],
        out_specs=[row, col],
        out_shape=[jax.ShapeDtypeStruct((N, L), jnp.float32),
                   jax.ShapeDtypeStruct((N, 1), jnp.float32)],
    )(emax, emin, wm1e, wm2, wua, wiht, wmsth)


def _select_body(mst_ref, pt_ref, out_ref):
    mst = mst_ref[...]
    pt = pt_ref[...]
    nt = jnp.where(pt != 0, _NEG, mst)
    iota = lax.broadcasted_iota(jnp.int32, (G, N // G), 1)
    rowmax = jnp.max(nt, axis=1, keepdims=True)
    cand = jnp.where(nt == rowmax, iota, jnp.int32(2**30))
    chosen = jnp.min(cand, axis=1, keepdims=True)
    out_ref[...] = jnp.where(iota == chosen, 1.0, pt)


def _select(mst_g, pt_g):
    blk = pl.BlockSpec((G, N // G), lambda: (0, 0))
    return pl.pallas_call(
        _select_body,
        in_specs=[blk, blk],
        out_specs=blk,
        out_shape=jax.ShapeDtypeStruct((G, N // G), jnp.float32),
    )(mst_g, pt_g)


def _final_body(enc_ref, raw_ref, h_ref, wue_ref, wua_ref, wiht_ref,
                whht_ref, wp1s_ref, wp1d_ref, c_ref, d_ref):
    raw = raw_ref[...]
    aggr = jnp.where(raw > -jnp.inf, raw, 0.0)
    enc = enc_ref[...]
    h = h_ref[...]
    u = _leaky(jnp.dot(enc, wue_ref[...], preferred_element_type=jnp.float32) +
               jnp.dot(aggr, wua_ref[...], preferred_element_type=jnp.float32))
    gh = jnp.dot(h, whht_ref[...], preferred_element_type=jnp.float32)
    hn = _gru(u, gh, h, wiht_ref)
    c_ref[...] = jnp.dot(hn, wp1s_ref[...], preferred_element_type=jnp.float32)
    d_ref[...] = jnp.dot(hn, wp1d_ref[...], preferred_element_type=jnp.float32)


def _final_update(enc, raw, h, wue, wua, wiht, whht, wp1s, wp1d):
    row = pl.BlockSpec((BN, L), lambda i: (i, 0))
    full = pl.BlockSpec((L, L), lambda i: (0, 0))
    full3 = pl.BlockSpec((L, 3 * L), lambda i: (0, 0))
    return pl.pallas_call(
        _final_body,
        grid=(NBLK,),
        in_specs=[row, row, row, full, full, full3, full3, full, full],
        out_specs=[row, row],
        out_shape=[jax.ShapeDtypeStruct((N, L), jnp.float32)] * 2,
    )(enc, raw, h, wue, wua, wiht, whht, wp1s, wp1d)


def _pout_body(c_ref, d_ref, ea_ref, w_ref, wp2_ref, o_ref):
    pin = c_ref[...] + d_ref[...] + ea_ref[...] * w_ref[...]
    o_ref[...] = jnp.dot(jax.nn.relu(pin), wp2_ref[...],
                         preferred_element_type=jnp.float32)


def _pout(csrc, ddst, ea2, wp1e, wp2):
    row = pl.BlockSpec((BE, L), lambda i: (i, 0))
    col = pl.BlockSpec((BE, 1), lambda i: (i, 0))
    vec = pl.BlockSpec((1, L), lambda i: (0, 0))
    cvec = pl.BlockSpec((L, 1), lambda i: (0, 0))
    return pl.pallas_call(
        _pout_body,
        grid=(EBLK,),
        in_specs=[row, row, col, vec, cvec],
        out_specs=col,
        out_shape=jax.ShapeDtypeStruct((E, 1), jnp.float32),
    )(csrc, ddst, ea2, wp1e, wp2)


def kernel(x, edge_attr, edge_index, W_enc, W_m1, W_m2, W_u, W_ih, W_hh, W_mst, W_p1, W_p2):
    n = x.shape[0]
    steps = x.shape[1]
    src = edge_index[0]
    dst = edge_index[1]
    ea = edge_attr
    ea2 = ea[:, None]

    w_enc0 = W_enc[0:1]
    W_enc1 = W_enc[1:]
    Wm1_d = W_m1[:L]
    Wm1_s = W_m1[L:2 * L]
    wm1_e = W_m1[2 * L:2 * L + 1]
    Wu_e = W_u[:L]
    Wu_a = W_u[L:]
    W_ihT = W_ih.T
    W_hhT = W_hh.T
    Wmst_e = W_mst[:L]
    Wmst_h = W_mst[L:]
    Wp1_s = W_p1[:L]
    Wp1_d = W_p1[L:2 * L]
    wp1_e = W_p1[2 * L:2 * L + 1]

    pt = x[:, 0]

    did, dcnt, addr, spid, scnt = _sc_prep(dst, src)

    ea_max = jax.ops.segment_max(ea, dst, num_segments=n)[:, None]
    ea_min = jax.ops.segment_min(ea, dst, num_segments=n)[:, None]
    h, mst = _update0(ea_max, ea_min, wm1_e, W_m2, Wu_a, W_ihT, Wmst_h)
    pt = _select(mst.reshape(G, n // G), pt.reshape(G, n // G)).reshape(-1)

    for step in range(1, steps):
        enc, A, B = _prep(pt[:, None], h, w_enc0, W_enc1, Wm1_d, Wm1_s)
        adst, bsrc = _sc_gather2(A, B, dst, src)
        m = _mmlp(adst, bsrc, ea2, wm1_e, W_m2)
        raw = _sc_segmax(m, did, dcnt)
        if step < steps - 1:
            h, mst = _update(enc, raw, h, Wu_e, Wu_a, W_ihT, W_hhT,
                             Wmst_e, Wmst_h)
            pt = _select(mst.reshape(G, n // G),
                         pt.reshape(G, n // G)).reshape(-1)
        else:
            C, D = _final_update(enc, raw, h, Wu_e, Wu_a, W_ihT, W_hhT,
                                 Wp1_s, Wp1_d)

    csrc, ddst = _sc_gather2(C, D, src, dst)
    p_out = _pout(csrc, ddst, ea2, wp1_e, W_p2)[:, 0]
    pred_flat = _sc_pred(p_out, addr, spid, scnt)
    return pred_flat[:n * n].reshape(n, n)


# speedup vs baseline: 1.0735x; 1.0010x over previous
import functools

import jax
import jax.numpy as jnp
from jax import lax
from jax.experimental import pallas as pl
from jax.experimental.pallas import tpu as pltpu
from jax.experimental.pallas import tpu_sc as plsc

G = 16
N = 4096
E = 131072
L = 128

NBLK = 8
BN = N // NBLK
EBLK = 128
BE = E // EBLK

_NEG = -1e9


def _leaky(v):
    return jnp.where(v >= 0, v, 0.01 * v)


_NC, _NS = 2, 16
_SC_PARAMS = pltpu.CompilerParams(needs_layout_passes=False)
_NW = _NC * _NS
_EPW = E // _NW
_GCH = 256
_NCH = _EPW // _GCH


def _sc_gather2_body(a_hbm, b_hbm, dst_hbm, src_hbm, adst_hbm, bsrc_hbm,
                     idx0_v, idx1_v, rows0_v, rows1_v, sem0, sem1):
    wid = lax.axis_index("s") * _NC + lax.axis_index("c")
    base0 = wid * _EPW
    idxs = (idx0_v, idx1_v)
    rows = (rows0_v, rows1_v)
    sems = (sem0, sem1)

    def task_refs(j):
        half = j // _NCH
        tab = a_hbm if half == 0 else b_hbm
        ind = dst_hbm if half == 0 else src_hbm
        out = adst_hbm if half == 0 else bsrc_hbm
        off = base0 + (j % _NCH) * _GCH
        return tab, ind, out, off

    tab, ind, out, off = task_refs(0)
    pltpu.sync_copy(ind.at[pl.ds(off, _GCH)], idxs[0])
    pltpu.async_copy(tab.at[idxs[0]], rows[0], sems[0])

    for j in range(2 * _NCH):
        b = j % 2
        nb = (j + 1) % 2
        if j + 1 < 2 * _NCH:
            tab, ind, out, off = task_refs(j + 1)
            pltpu.sync_copy(ind.at[pl.ds(off, _GCH)], idxs[nb])
            pltpu.async_copy(tab.at[idxs[nb]], rows[nb], sems[nb])
        tab, ind, out, off = task_refs(j)
        pltpu.make_async_copy(tab.at[idxs[b]], rows[b], sems[b]).wait()
        pltpu.sync_copy(rows[b], out.at[pl.ds(off, _GCH)])


def _sc_gather2(a, b, dst, src):
    mesh = plsc.VectorSubcoreMesh(core_axis_name="c", subcore_axis_name="s")
    f = pl.kernel(
        _sc_gather2_body,
        mesh=mesh,
        compiler_params=_SC_PARAMS,
        out_type=(
            jax.ShapeDtypeStruct((E, L), jnp.float32),
            jax.ShapeDtypeStruct((E, L), jnp.float32),
        ),
        scratch_types=[
            pltpu.VMEM((_GCH,), jnp.int32),
            pltpu.VMEM((_GCH,), jnp.int32),
            pltpu.VMEM((_GCH, L), jnp.float32),
            pltpu.VMEM((_GCH, L), jnp.float32),
            pltpu.SemaphoreType.DMA,
            pltpu.SemaphoreType.DMA,
        ],
    )
    return f(a, b, dst, src)


CSEG = 256
_FB = 1024
ECAP = E + CSEG
_NPW = N // _NW
_DUMP = _NPW * 8
PREDPAD = 16 * _NW


def _append_flush(buf_refs, hbm_refs, vals, mask, cnt, nf, wbase):
    for br, v in zip(buf_refs, vals):
        plsc.store_compressed(br.at[pl.ds(cnt, 16)], v, mask=mask)
    cnt = cnt + jnp.sum(mask.astype(jnp.int32))

    def flush():
        for br, hr in zip(buf_refs, hbm_refs):
            pltpu.sync_copy(br.at[pl.ds(0, _FB)],
                            hr.at[pl.ds(wbase + nf * _FB, _FB)])
            rem = br[pl.ds(_FB, 16)]
            br[pl.ds(0, 16)] = rem

    jax.lax.cond(cnt >= _FB, flush, lambda: None)
    new_nf = jnp.where(cnt >= _FB, nf + 1, nf)
    new_cnt = jnp.where(cnt >= _FB, cnt - _FB, cnt)
    return new_cnt, new_nf


def _pad_tail(buf_refs, hbm_refs, pads, cnt, nf, wbase):
    base16 = (cnt // 16) * 16
    lanes = lax.iota(jnp.int32, 16)
    for br, padv in zip(buf_refs, pads):
        cur = br[pl.ds(base16, 16)]
        br[pl.ds(base16, 16)] = jnp.where(base16 + lanes < cnt, cur, padv)
        for k in range(1, 18):
            br[pl.ds(base16 + k * 16, 16)] = jnp.zeros((16,), jnp.int32) + padv
    padded = ((cnt + CSEG - 1) // CSEG) * CSEG

    def flush_k(k, _):
        for br, hr in zip(buf_refs, hbm_refs):
            pltpu.sync_copy(br.at[pl.ds(k * CSEG, CSEG)],
                            hr.at[pl.ds(wbase + nf * _FB + k * CSEG, CSEG)])
        return ()

    lax.fori_loop(0, padded // CSEG, flush_k, ())
    return nf * _FB + padded


_PCH = 4096


def _sc_prep_body(dst_hbm, src_hbm,
                  did_hbm, dcnt_hbm, addr_hbm, spid_hbm, scnt_hbm,
                  d_v, s_v, did_v, addr_v, spid_v, cnt_v):
    wid = lax.axis_index("s") * _NC + lax.axis_index("c")
    lo = wid * _NPW
    wbase = wid * ECAP

    def chunk(i, carry):
        cnt1, nf1, cnt2, nf2 = carry
        pltpu.sync_copy(dst_hbm.at[pl.ds(i * _PCH, _PCH)], d_v)
        pltpu.sync_copy(src_hbm.at[pl.ds(i * _PCH, _PCH)], s_v)

        def vreg(j, carry2):
            c1, n1, c2, n2 = carry2
            d = d_v[pl.ds(j * 16, 16)]
            s = s_v[pl.ds(j * 16, 16)]
            ids = lax.iota(jnp.int32, 16) + (i * _PCH + j * 16)
            mask_d = (d >= lo) & (d < lo + _NPW)
            packed = ids | ((d - lo) << 18)
            c1, n1 = _append_flush((did_v,), (did_hbm,),
                                   (packed,), mask_d, c1, n1, wbase)
            mask_s = (s >= lo) & (s < lo + _NPW)
            addr = s * N + d
            c2, n2 = _append_flush((addr_v, spid_v), (addr_hbm, spid_hbm),
                                   (addr, ids), mask_s, c2, n2, wbase)
            return c1, n1, c2, n2

        return lax.fori_loop(0, _PCH // 16, vreg, (cnt1, nf1, cnt2, nf2),
                             unroll=2)

    cnt1, nf1, cnt2, nf2 = lax.fori_loop(
        0, E // _PCH, chunk,
        (jnp.int32(0), jnp.int32(0), jnp.int32(0), jnp.int32(0)))

    tot1 = _pad_tail((did_v,), (did_hbm,),
                     (jnp.int32(_NPW << 18),), cnt1, nf1, wbase)
    tot2 = _pad_tail((addr_v, spid_v), (addr_hbm, spid_hbm),
                     (jnp.int32(N * N) + wid * 16, jnp.int32(0)),
                     cnt2, nf2, wbase)
    cnt_v[...] = jnp.zeros((16,), jnp.int32) + tot1
    pltpu.sync_copy(cnt_v, dcnt_hbm.at[pl.ds(wid * 16, 16)])
    cnt_v[...] = jnp.zeros((16,), jnp.int32) + tot2
    pltpu.sync_copy(cnt_v, scnt_hbm.at[pl.ds(wid * 16, 16)])


def _sc_prep(dst, src):
    mesh = plsc.VectorSubcoreMesh(core_axis_name="c", subcore_axis_name="s")
    lbuf = pltpu.VMEM((_FB + 16 + 288,), jnp.int32)
    f = pl.kernel(
        _sc_prep_body,
        mesh=mesh,
        compiler_params=_SC_PARAMS,
        out_type=(
            jax.ShapeDtypeStruct((_NW * ECAP,), jnp.int32),
            jax.ShapeDtypeStruct((_NW * 16,), jnp.int32),
            jax.ShapeDtypeStruct((_NW * ECAP,), jnp.int32),
            jax.ShapeDtypeStruct((_NW * ECAP,), jnp.int32),
            jax.ShapeDtypeStruct((_NW * 16,), jnp.int32),
        ),
        scratch_types=[
            pltpu.VMEM((_PCH,), jnp.int32),
            pltpu.VMEM((_PCH,), jnp.int32),
            lbuf, lbuf, lbuf,
            pltpu.VMEM((16,), jnp.int32),
        ],
    )
    return f(dst, src)


def _sc_segmax_body(m_hbm, did_hbm, dcnt_hbm, aggr_hbm,
                    pk0_v, pk1_v, ids0_v, ids1_v, rows0_v, rows1_v,
                    acc0_v, acc1_v,
                    cnt_v, lsem0, lsem1, gsem0, gsem1):
    accs = (acc0_v, acc1_v)
    pks = (pk0_v, pk1_v)
    idss = (ids0_v, ids1_v)
    rows = (rows0_v, rows1_v)
    lsems = (lsem0, lsem1)
    gsems = (gsem0, gsem1)
    wid = lax.axis_index("s") * _NC + lax.axis_index("c")
    wbase = wid * ECAP

    def initrow(j, _):
        for a in accs:
            for c in range(8):
                a[j, pl.ds(c * 16, 16)] = jnp.full((16,), -jnp.inf, jnp.float32)
        return ()

    lax.fori_loop(0, _NPW + 1, initrow, ())

    pltpu.sync_copy(dcnt_hbm.at[pl.ds(wid * 16, 16)], cnt_v)
    trips = cnt_v[...][0] // CSEG

    def stage(t, b):
        pltpu.make_async_copy(did_hbm.at[pl.ds(wbase + t * CSEG, CSEG)],
                              pks[b], lsems[b]).wait()
        for g in range(CSEG // 16):
            p = pks[b][pl.ds(g * 16, 16)]
            idss[b][pl.ds(g * 16, 16)] = p & ((1 << 18) - 1)
        pltpu.async_copy(m_hbm.at[idss[b]], rows[b], gsems[b])

    def compute(b):
        def group(g, _):
            dlv = pks[b][pl.ds(g * 16, 16)] >> 18
            for k in range(16):
                e = g * 16 + k
                acc = accs[k % 2]
                dl = dlv[k]
                for c in range(8):
                    r = rows[b][e, pl.ds(c * 16, 16)]
                    acc[dl, pl.ds(c * 16, 16)] = jnp.maximum(
                        acc[dl, pl.ds(c * 16, 16)], r)
            return ()

        lax.fori_loop(0, CSEG // 16, group, ())

    @pl.when(trips > 0)
    def _():
        pltpu.async_copy(did_hbm.at[pl.ds(wbase, CSEG)], pks[0], lsems[0])
        stage(0, 0)

    def chunk(t, _):
        def do(b, nb):
            @pl.when(t + 1 < trips)
            def _():
                pltpu.async_copy(
                    did_hbm.at[pl.ds(wbase + (t + 1) * CSEG, CSEG)],
                    pks[nb], lsems[nb])
                stage(t + 1, nb)
            pltpu.make_async_copy(m_hbm.at[idss[b]], rows[b], gsems[b]).wait()
            compute(b)

        @pl.when(t % 2 == 0)
        def _():
            do(0, 1)

        @pl.when(t % 2 == 1)
        def _():
            do(1, 0)

        return ()

    lax.fori_loop(0, trips, chunk, ())

    def mergerow(j, _):
        for c in range(8):
            acc0_v[j, pl.ds(c * 16, 16)] = jnp.maximum(
                acc0_v[j, pl.ds(c * 16, 16)],
                acc1_v[j, pl.ds(c * 16, 16)])
        return ()

    lax.fori_loop(0, _NPW, mergerow, ())
    pltpu.sync_copy(acc0_v.at[pl.ds(0, _NPW)],
                    aggr_hbm.at[pl.ds(wid * _NPW, _NPW)])


def _sc_segmax(m, did, dcnt):
    mesh = plsc.VectorSubcoreMesh(core_axis_name="c", subcore_axis_name="s")
    f = pl.kernel(
        _sc_segmax_body,
        mesh=mesh,
        compiler_params=_SC_PARAMS,
        out_type=jax.ShapeDtypeStruct((N, L), jnp.float32),
        scratch_types=[
            pltpu.VMEM((CSEG,), jnp.int32),
            pltpu.VMEM((CSEG,), jnp.int32),
            pltpu.VMEM((CSEG,), jnp.int32),
            pltpu.VMEM((CSEG,), jnp.int32),
            pltpu.VMEM((CSEG, L), jnp.float32),
            pltpu.VMEM((CSEG, L), jnp.float32),
            pltpu.VMEM((_NPW + 1, L), jnp.float32),
            pltpu.VMEM((_NPW + 1, L), jnp.float32),
            pltpu.VMEM((16,), jnp.int32),
            pltpu.SemaphoreType.DMA,
            pltpu.SemaphoreType.DMA,
            pltpu.SemaphoreType.DMA,
            pltpu.SemaphoreType.DMA,
        ],
    )
    return f(m, did, dcnt)


_FILLB = 16384
_RPW = N * N // _NW


def _sc_pred_body(pout_hbm, addr_hbm, spid_hbm, scnt_hbm, pred_hbm,
                  fill_v, a0_v, a1_v, p0_v, p1_v, v0_v, v1_v, cnt_v,
                  sem, la0, la1, lp0, lp1, g0, g1):
    addrs = (a0_v, a1_v)
    pids = (p0_v, p1_v)
    vals = (v0_v, v1_v)
    lasems = (la0, la1)
    lpsems = (lp0, lp1)
    gsems = (g0, g1)
    wid = lax.axis_index("s") * _NC + lax.axis_index("c")
    wbase = wid * ECAP

    def initf(j, _):
        fill_v[pl.ds(j * 16, 16)] = jnp.full((16,), _NEG, jnp.float32)
        return ()

    lax.fori_loop(0, _FILLB // 16, initf, ())

    def fill(k, _):
        pltpu.async_copy(fill_v,
                         pred_hbm.at[pl.ds(wid * _RPW + k * _FILLB, _FILLB)],
                         sem)
        return ()

    lax.fori_loop(0, _RPW // _FILLB, fill, ())

    def fill_wait(k, _):
        pltpu.make_async_copy(
            fill_v, pred_hbm.at[pl.ds(wid * _RPW + k * _FILLB, _FILLB)],
            sem).wait()
        return ()

    lax.fori_loop(0, _RPW // _FILLB, fill_wait, ())

    pltpu.sync_copy(scnt_hbm.at[pl.ds(wid * 16, 16)], cnt_v)
    trips = cnt_v[...][0] // CSEG

    def prefetch(t, b):
        pltpu.async_copy(addr_hbm.at[pl.ds(wbase + t * CSEG, CSEG)],
                         addrs[b], lasems[b])
        pltpu.async_copy(spid_hbm.at[pl.ds(wbase + t * CSEG, CSEG)],
                         pids[b], lpsems[b])

    def stage(t, b):
        pltpu.make_async_copy(addr_hbm.at[pl.ds(wbase + t * CSEG, CSEG)],
                              addrs[b], lasems[b]).wait()
        pltpu.make_async_copy(spid_hbm.at[pl.ds(wbase + t * CSEG, CSEG)],
                              pids[b], lpsems[b]).wait()
        pltpu.async_copy(pout_hbm.at[pids[b]], vals[b], gsems[b])

    @pl.when(trips > 0)
    def _():
        prefetch(0, 0)
        stage(0, 0)

    def chunk(t, _):
        def do(b, nb):
            @pl.when(t + 1 < trips)
            def _():
                prefetch(t + 1, nb)
                stage(t + 1, nb)
            pltpu.make_async_copy(pout_hbm.at[pids[b]], vals[b],
                                  gsems[b]).wait()
            pltpu.async_copy(vals[b], pred_hbm.at[addrs[b]], sem).wait()

        @pl.when(t % 2 == 0)
        def _():
            do(0, 1)

        @pl.when(t % 2 == 1)
        def _():
            do(1, 0)

        return ()

    lax.fori_loop(0, trips, chunk, ())


def _sc_pred(pout, addr, spid, scnt):
    mesh = plsc.VectorSubcoreMesh(core_axis_name="c", subcore_axis_name="s")
    f = pl.kernel(
        _sc_pred_body,
        mesh=mesh,
        compiler_params=_SC_PARAMS,
        out_type=jax.ShapeDtypeStruct((N * N + PREDPAD,), jnp.float32),
        scratch_types=[
            pltpu.VMEM((_FILLB,), jnp.float32),
            pltpu.VMEM((CSEG,), jnp.int32),
            pltpu.VMEM((CSEG,), jnp.int32),
            pltpu.VMEM((CSEG,), jnp.int32),
            pltpu.VMEM((CSEG,), jnp.int32),
            pltpu.VMEM((CSEG,), jnp.float32),
            pltpu.VMEM((CSEG,), jnp.float32),
            pltpu.VMEM((16,), jnp.int32),
            pltpu.SemaphoreType.DMA,
            pltpu.SemaphoreType.DMA,
            pltpu.SemaphoreType.DMA,
            pltpu.SemaphoreType.DMA,
            pltpu.SemaphoreType.DMA,
            pltpu.SemaphoreType.DMA,
            pltpu.SemaphoreType.DMA,
        ],
    )
    return f(pout, addr, spid, scnt)


def _prep_body(pt_ref, h_ref, w0_ref, wenc1_ref, wm1d_ref, wm1s_ref,
               enc_ref, a_ref, b_ref):
    enc = jax.nn.relu(pt_ref[...] * w0_ref[...] +
                      jnp.dot(h_ref[...], wenc1_ref[...],
                              preferred_element_type=jnp.float32))
    enc_ref[...] = enc
    a_ref[...] = jnp.dot(enc, wm1d_ref[...], preferred_element_type=jnp.float32)
    b_ref[...] = jnp.dot(enc, wm1s_ref[...], preferred_element_type=jnp.float32)


def _prep(pt, h, w0, wenc1, wm1d, wm1s):
    row = pl.BlockSpec((BN, L), lambda i: (i, 0))
    col = pl.BlockSpec((BN, 1), lambda i: (i, 0))
    full = pl.BlockSpec((L, L), lambda i: (0, 0))
    vec = pl.BlockSpec((1, L), lambda i: (0, 0))
    return pl.pallas_call(
        _prep_body,
        grid=(NBLK,),
        in_specs=[col, row, vec, full, full, full],
        out_specs=[row, row, row],
        out_shape=[jax.ShapeDtypeStruct((N, L), jnp.float32)] * 3,
    )(pt, h, w0, wenc1, wm1d, wm1s)


def _mmlp_body(a_ref, b_ref, ea_ref, w_ref, wm2_ref, o_ref):
    pre = a_ref[...] + b_ref[...] + ea_ref[...] * w_ref[...]
    q = _leaky(pre)
    o_ref[...] = _leaky(jnp.dot(q, wm2_ref[...],
                                preferred_element_type=jnp.float32))


def _mmlp(adst, bsrc, ea2, w, wm2):
    row = pl.BlockSpec((BE, L), lambda i: (i, 0))
    col = pl.BlockSpec((BE, 1), lambda i: (i, 0))
    full = pl.BlockSpec((L, L), lambda i: (0, 0))
    vec = pl.BlockSpec((1, L), lambda i: (0, 0))
    return pl.pallas_call(
        _mmlp_body,
        grid=(EBLK,),
        in_specs=[row, row, col, vec, full],
        out_specs=row,
        out_shape=jax.ShapeDtypeStruct((E, L), jnp.float32),
    )(adst, bsrc, ea2, w, wm2)


def _gru(u, gh_ref, h, w_iht_ref):
    gi = jnp.dot(u, w_iht_ref[...], preferred_element_type=jnp.float32)
    gh = gh_ref
    r = jax.nn.sigmoid(gi[:, :L] + gh[:, :L])
    z = jax.nn.sigmoid(gi[:, L:2 * L] + gh[:, L:2 * L])
    ng = jnp.tanh(gi[:, 2 * L:] + r * gh[:, 2 * L:])
    return (1.0 - z) * ng + z * h


def _update_body(enc_ref, raw_ref, h_ref, wue_ref, wua_ref, wiht_ref,
                 whht_ref, wmste_ref, wmsth_ref, hn_ref, mst_ref):
    raw = raw_ref[...]
    aggr = jnp.where(raw > -jnp.inf, raw, 0.0)
    enc = enc_ref[...]
    h = h_ref[...]
    u = _leaky(jnp.dot(enc, wue_ref[...], preferred_element_type=jnp.float32) +
               jnp.dot(aggr, wua_ref[...], preferred_element_type=jnp.float32))
    gh = jnp.dot(h, whht_ref[...], preferred_element_type=jnp.float32)
    hn = _gru(u, gh, h, wiht_ref)
    hn_ref[...] = hn
    mst_ref[...] = (jnp.dot(enc, wmste_ref[...], preferred_element_type=jnp.float32) +
                    jnp.dot(hn, wmsth_ref[...], preferred_element_type=jnp.float32))


def _update(enc, raw, h, wue, wua, wiht, whht, wmste, wmsth):
    row = pl.BlockSpec((BN, L), lambda i: (i, 0))
    full = pl.BlockSpec((L, L), lambda i: (0, 0))
    full3 = pl.BlockSpec((L, 3 * L), lambda i: (0, 0))
    cvec = pl.BlockSpec((L, 1), lambda i: (0, 0))
    col = pl.BlockSpec((BN, 1), lambda i: (i, 0))
    return pl.pallas_call(
        _update_body,
        grid=(NBLK,),
        in_specs=[row, row, row, full, full, full3, full3, cvec, cvec],
        out_specs=[row, col],
        out_shape=[jax.ShapeDtypeStruct((N, L), jnp.float32),
                   jax.ShapeDtypeStruct((N, 1), jnp.float32)],
    )(enc, raw, h, wue, wua, wiht, whht, wmste, wmsth)


def _update0_body(emax_ref, emin_ref, wm1e_ref, wm2_ref, wua_ref, wiht_ref,
                  wmsth_ref, hn_ref, mst_ref):
    g = _leaky(jnp.dot(_leaky(wm1e_ref[...]), wm2_ref[...],
                       preferred_element_type=jnp.float32))
    emax_raw = emax_ref[...]
    emin_raw = emin_ref[...]
    emax = jnp.where(emax_raw > -jnp.inf, emax_raw, 0.0)
    emin = jnp.where(emin_raw < jnp.inf, emin_raw, 0.0)
    aggr = jnp.where(g > 0, emax * g, emin * g)
    u = _leaky(jnp.dot(aggr, wua_ref[...], preferred_element_type=jnp.float32))
    gi = jnp.dot(u, wiht_ref[...], preferred_element_type=jnp.float32)
    z = jax.nn.sigmoid(gi[:, L:2 * L])
    ng = jnp.tanh(gi[:, 2 * L:])
    hn = (1.0 - z) * ng
    hn_ref[...] = hn
    mst_ref[...] = jnp.dot(hn, wmsth_ref[...], preferred_element_type=jnp.float32)


def _update0(emax, emin, wm1e, wm2, wua, wiht, wmsth):
    row = pl.BlockSpec((BN, L), lambda i: (i, 0))
    col = pl.BlockSpec((BN, 1), lambda i: (i, 0))
    full = pl.BlockSpec((L, L), lambda i: (0, 0))
    full3 = pl.BlockSpec((L, 3 * L), lambda i: (0, 0))
    vec = pl.BlockSpec((1, L), lambda i: (0, 0))
    cvec = pl.BlockSpec((L, 1), lambda i: (0, 0))
    return pl.pallas_call(
        _update0_body,
        grid=(NBLK,),
        in_specs=[col, col, vec, full, full, full3, cvec],
        out_specs=[row, col],
        out_shape=[jax.ShapeDtypeStruct((N, L), jnp.float32),
                   jax.ShapeDtypeStruct((N, 1), jnp.float32)],
    )(emax, emin, wm1e, wm2, wua, wiht, wmsth)


def _select_body(mst_ref, pt_ref, out_ref):
    mst = mst_ref[...]
    pt = pt_ref[...]
    nt = jnp.where(pt != 0, _NEG, mst)
    iota = lax.broadcasted_iota(jnp.int32, (G, N // G), 1)
    rowmax = jnp.max(nt, axis=1, keepdims=True)
    cand = jnp.where(nt == rowmax, iota, jnp.int32(2**30))
    chosen = jnp.min(cand, axis=1, keepdims=True)
    out_ref[...] = jnp.where(iota == chosen, 1.0, pt)


def _select(mst_g, pt_g):
    blk = pl.BlockSpec((G, N // G), lambda: (0, 0))
    return pl.pallas_call(
        _select_body,
        in_specs=[blk, blk],
        out_specs=blk,
        out_shape=jax.ShapeDtypeStruct((G, N // G), jnp.float32),
    )(mst_g, pt_g)


def _final_body(enc_ref, raw_ref, h_ref, wue_ref, wua_ref, wiht_ref,
                whht_ref, wp1s_ref, wp1d_ref, c_ref, d_ref):
    raw = raw_ref[...]
    aggr = jnp.where(raw > -jnp.inf, raw, 0.0)
    enc = enc_ref[...]
    h = h_ref[...]
    u = _leaky(jnp.dot(enc, wue_ref[...], preferred_element_type=jnp.float32) +
               jnp.dot(aggr, wua_ref[...], preferred_element_type=jnp.float32))
    gh = jnp.dot(h, whht_ref[...], preferred_element_type=jnp.float32)
    hn = _gru(u, gh, h, wiht_ref)
    c_ref[...] = jnp.dot(hn, wp1s_ref[...], preferred_element_type=jnp.float32)
    d_ref[...] = jnp.dot(hn, wp1d_ref[...], preferred_element_type=jnp.float32)


def _final_update(enc, raw, h, wue, wua, wiht, whht, wp1s, wp1d):
    row = pl.BlockSpec((BN, L), lambda i: (i, 0))
    full = pl.BlockSpec((L, L), lambda i: (0, 0))
    full3 = pl.BlockSpec((L, 3 * L), lambda i: (0, 0))
    return pl.pallas_call(
        _final_body,
        grid=(NBLK,),
        in_specs=[row, row, row, full, full, full3, full3, full, full],
        out_specs=[row, row],
        out_shape=[jax.ShapeDtypeStruct((N, L), jnp.float32)] * 2,
    )(enc, raw, h, wue, wua, wiht, whht, wp1s, wp1d)


def _pout_body(c_ref, d_ref, ea_ref, w_ref, wp2_ref, o_ref):
    pin = c_ref[...] + d_ref[...] + ea_ref[...] * w_ref[...]
    o_ref[...] = jnp.dot(jax.nn.relu(pin), wp2_ref[...],
                         preferred_element_type=jnp.float32)


def _pout(csrc, ddst, ea2, wp1e, wp2):
    row = pl.BlockSpec((BE, L), lambda i: (i, 0))
    col = pl.BlockSpec((BE, 1), lambda i: (i, 0))
    vec = pl.BlockSpec((1, L), lambda i: (0, 0))
    cvec = pl.BlockSpec((L, 1), lambda i: (0, 0))
    return pl.pallas_call(
        _pout_body,
        grid=(EBLK,),
        in_specs=[row, row, col, vec, cvec],
        out_specs=col,
        out_shape=jax.ShapeDtypeStruct((E, 1), jnp.float32),
    )(csrc, ddst, ea2, wp1e, wp2)


def kernel(x, edge_attr, edge_index, W_enc, W_m1, W_m2, W_u, W_ih, W_hh, W_mst, W_p1, W_p2):
    n = x.shape[0]
    steps = x.shape[1]
    src = edge_index[0]
    dst = edge_index[1]
    ea = edge_attr
    ea2 = ea[:, None]

    w_enc0 = W_enc[0:1]
    W_enc1 = W_enc[1:]
    Wm1_d = W_m1[:L]
    Wm1_s = W_m1[L:2 * L]
    wm1_e = W_m1[2 * L:2 * L + 1]
    Wu_e = W_u[:L]
    Wu_a = W_u[L:]
    W_ihT = W_ih.T
    W_hhT = W_hh.T
    Wmst_e = W_mst[:L]
    Wmst_h = W_mst[L:]
    Wp1_s = W_p1[:L]
    Wp1_d = W_p1[L:2 * L]
    wp1_e = W_p1[2 * L:2 * L + 1]

    pt = x[:, 0]

    did, dcnt, addr, spid, scnt = _sc_prep(dst, src)

    ea_max = jax.ops.segment_max(ea, dst, num_segments=n)[:, None]
    ea_min = jax.ops.segment_min(ea, dst, num_segments=n)[:, None]
    h, mst = _update0(ea_max, ea_min, wm1_e, W_m2, Wu_a, W_ihT, Wmst_h)
    pt = _select(mst.reshape(G, n // G), pt.reshape(G, n // G)).reshape(-1)

    for step in range(1, steps):
        enc, A, B = _prep(pt[:, None], h, w_enc0, W_enc1, Wm1_d, Wm1_s)
        adst, bsrc = _sc_gather2(A, B, dst, src)
        m = _mmlp(adst, bsrc, ea2, wm1_e, W_m2)
        raw = _sc_segmax(m, did, dcnt)
        if step < steps - 1:
            h, mst = _update(enc, raw, h, Wu_e, Wu_a, W_ihT, W_hhT,
                             Wmst_e, Wmst_h)
            pt = _select(mst.reshape(G, n // G),
                         pt.reshape(G, n // G)).reshape(-1)
        else:
            C, D = _final_update(enc, raw, h, Wu_e, Wu_a, W_ihT, W_hhT,
                                 Wp1_s, Wp1_d)

    csrc, ddst = _sc_gather2(C, D, src, dst)
    p_out = _pout(csrc, ddst, ea2, wp1_e, W_p2)[:, 0]
    pred_flat = _sc_pred(p_out, addr, spid, scnt)
    return pred_flat[:n * n].reshape(n, n)


# speedup vs baseline: 1.0854x; 1.0111x over previous
import functools

import jax
import jax.numpy as jnp
from jax import lax
from jax.experimental import pallas as pl
from jax.experimental.pallas import tpu as pltpu
from jax.experimental.pallas import tpu_sc as plsc

G = 16
N = 4096
E = 131072
L = 128

NBLK = 8
BN = N // NBLK
EBLK = 128
BE = E // EBLK

_NEG = -1e9


def _leaky(v):
    return jnp.where(v >= 0, v, 0.01 * v)


_NC, _NS = 2, 16
_SC_PARAMS = pltpu.CompilerParams(needs_layout_passes=False)
_NW = _NC * _NS
_EPW = E // _NW
_GCH = 256
_NCH = _EPW // _GCH


def _sc_gather2_body(a_hbm, b_hbm, dst_hbm, src_hbm, adst_hbm, bsrc_hbm,
                     idx0_v, idx1_v, rows0_v, rows1_v, sem0, sem1):
    wid = lax.axis_index("s") * _NC + lax.axis_index("c")
    base0 = wid * _EPW
    idxs = (idx0_v, idx1_v)
    rows = (rows0_v, rows1_v)
    sems = (sem0, sem1)

    def task_refs(j):
        half = j // _NCH
        tab = a_hbm if half == 0 else b_hbm
        ind = dst_hbm if half == 0 else src_hbm
        out = adst_hbm if half == 0 else bsrc_hbm
        off = base0 + (j % _NCH) * _GCH
        return tab, ind, out, off

    tab, ind, out, off = task_refs(0)
    pltpu.sync_copy(ind.at[pl.ds(off, _GCH)], idxs[0])
    pltpu.async_copy(tab.at[idxs[0]], rows[0], sems[0])

    for j in range(2 * _NCH):
        b = j % 2
        nb = (j + 1) % 2
        if j + 1 < 2 * _NCH:
            tab, ind, out, off = task_refs(j + 1)
            pltpu.sync_copy(ind.at[pl.ds(off, _GCH)], idxs[nb])
            pltpu.async_copy(tab.at[idxs[nb]], rows[nb], sems[nb])
        tab, ind, out, off = task_refs(j)
        pltpu.make_async_copy(tab.at[idxs[b]], rows[b], sems[b]).wait()
        pltpu.sync_copy(rows[b], out.at[pl.ds(off, _GCH)])


def _sc_gather2(a, b, dst, src):
    mesh = plsc.VectorSubcoreMesh(core_axis_name="c", subcore_axis_name="s")
    f = pl.kernel(
        _sc_gather2_body,
        mesh=mesh,
        compiler_params=_SC_PARAMS,
        out_type=(
            jax.ShapeDtypeStruct((E, L), jnp.float32),
            jax.ShapeDtypeStruct((E, L), jnp.float32),
        ),
        scratch_types=[
            pltpu.VMEM((_GCH,), jnp.int32),
            pltpu.VMEM((_GCH,), jnp.int32),
            pltpu.VMEM((_GCH, L), jnp.float32),
            pltpu.VMEM((_GCH, L), jnp.float32),
            pltpu.SemaphoreType.DMA,
            pltpu.SemaphoreType.DMA,
        ],
    )
    return f(a, b, dst, src)


CSEG = 256
_FB = 1024
ECAP = E + CSEG
_NPW = N // _NW
_DUMP = _NPW * 8
PREDPAD = 16 * _NW


def _append_flush(buf_refs, hbm_refs, vals, mask, cnt, nf, wbase):
    for br, v in zip(buf_refs, vals):
        plsc.store_compressed(br.at[pl.ds(cnt, 16)], v, mask=mask)
    cnt = cnt + jnp.sum(mask.astype(jnp.int32))

    def flush():
        for br, hr in zip(buf_refs, hbm_refs):
            pltpu.sync_copy(br.at[pl.ds(0, _FB)],
                            hr.at[pl.ds(wbase + nf * _FB, _FB)])
            rem = br[pl.ds(_FB, 16)]
            br[pl.ds(0, 16)] = rem

    jax.lax.cond(cnt >= _FB, flush, lambda: None)
    new_nf = jnp.where(cnt >= _FB, nf + 1, nf)
    new_cnt = jnp.where(cnt >= _FB, cnt - _FB, cnt)
    return new_cnt, new_nf


def _pad_tail(buf_refs, hbm_refs, pads, cnt, nf, wbase):
    base16 = (cnt // 16) * 16
    lanes = lax.iota(jnp.int32, 16)
    for br, padv in zip(buf_refs, pads):
        cur = br[pl.ds(base16, 16)]
        br[pl.ds(base16, 16)] = jnp.where(base16 + lanes < cnt, cur, padv)
        for k in range(1, 18):
            br[pl.ds(base16 + k * 16, 16)] = jnp.zeros((16,), jnp.int32) + padv
    padded = ((cnt + CSEG - 1) // CSEG) * CSEG

    def flush_k(k, _):
        for br, hr in zip(buf_refs, hbm_refs):
            pltpu.sync_copy(br.at[pl.ds(k * CSEG, CSEG)],
                            hr.at[pl.ds(wbase + nf * _FB + k * CSEG, CSEG)])
        return ()

    lax.fori_loop(0, padded // CSEG, flush_k, ())
    return nf * _FB + padded


_PCH = 16384


def _sc_prep_body(dst_hbm, src_hbm,
                  did_hbm, dcnt_hbm, addr_hbm, spid_hbm, scnt_hbm,
                  d_v, s_v, did_v, addr_v, spid_v, cnt_v):
    wid = lax.axis_index("s") * _NC + lax.axis_index("c")
    lo = wid * _NPW
    wbase = wid * ECAP

    def chunk(i, carry):
        cnt1, nf1, cnt2, nf2 = carry
        pltpu.sync_copy(dst_hbm.at[pl.ds(i * _PCH, _PCH)], d_v)
        pltpu.sync_copy(src_hbm.at[pl.ds(i * _PCH, _PCH)], s_v)

        def vreg(j, carry2):
            c1, n1, c2, n2 = carry2
            d = d_v[pl.ds(j * 16, 16)]
            s = s_v[pl.ds(j * 16, 16)]
            ids = lax.iota(jnp.int32, 16) + (i * _PCH + j * 16)
            mask_d = (d >= lo) & (d < lo + _NPW)
            packed = ids | ((d - lo) << 18)
            c1, n1 = _append_flush((did_v,), (did_hbm,),
                                   (packed,), mask_d, c1, n1, wbase)
            mask_s = (s >= lo) & (s < lo + _NPW)
            addr = s * N + d
            c2, n2 = _append_flush((addr_v, spid_v), (addr_hbm, spid_hbm),
                                   (addr, ids), mask_s, c2, n2, wbase)
            return c1, n1, c2, n2

        return lax.fori_loop(0, _PCH // 16, vreg, (cnt1, nf1, cnt2, nf2),
                             unroll=2)

    cnt1, nf1, cnt2, nf2 = lax.fori_loop(
        0, E // _PCH, chunk,
        (jnp.int32(0), jnp.int32(0), jnp.int32(0), jnp.int32(0)))

    tot1 = _pad_tail((did_v,), (did_hbm,),
                     (jnp.int32(_NPW << 18),), cnt1, nf1, wbase)
    tot2 = _pad_tail((addr_v, spid_v), (addr_hbm, spid_hbm),
                     (jnp.int32(N * N) + wid * 16, jnp.int32(0)),
                     cnt2, nf2, wbase)
    cnt_v[...] = jnp.zeros((16,), jnp.int32) + tot1
    pltpu.sync_copy(cnt_v, dcnt_hbm.at[pl.ds(wid * 16, 16)])
    cnt_v[...] = jnp.zeros((16,), jnp.int32) + tot2
    pltpu.sync_copy(cnt_v, scnt_hbm.at[pl.ds(wid * 16, 16)])


def _sc_prep(dst, src):
    mesh = plsc.VectorSubcoreMesh(core_axis_name="c", subcore_axis_name="s")
    lbuf = pltpu.VMEM((_FB + 16 + 288,), jnp.int32)
    f = pl.kernel(
        _sc_prep_body,
        mesh=mesh,
        compiler_params=_SC_PARAMS,
        out_type=(
            jax.ShapeDtypeStruct((_NW * ECAP,), jnp.int32),
            jax.ShapeDtypeStruct((_NW * 16,), jnp.int32),
            jax.ShapeDtypeStruct((_NW * ECAP,), jnp.int32),
            jax.ShapeDtypeStruct((_NW * ECAP,), jnp.int32),
            jax.ShapeDtypeStruct((_NW * 16,), jnp.int32),
        ),
        scratch_types=[
            pltpu.VMEM((_PCH,), jnp.int32),
            pltpu.VMEM((_PCH,), jnp.int32),
            lbuf, lbuf, lbuf,
            pltpu.VMEM((16,), jnp.int32),
        ],
    )
    return f(dst, src)


def _sc_segmax_body(m_hbm, did_hbm, dcnt_hbm, aggr_hbm,
                    pk0_v, pk1_v, ids0_v, ids1_v, rows0_v, rows1_v,
                    acc0_v, acc1_v,
                    cnt_v, lsem0, lsem1, gsem0, gsem1):
    accs = (acc0_v, acc1_v)
    pks = (pk0_v, pk1_v)
    idss = (ids0_v, ids1_v)
    rows = (rows0_v, rows1_v)
    lsems = (lsem0, lsem1)
    gsems = (gsem0, gsem1)
    wid = lax.axis_index("s") * _NC + lax.axis_index("c")
    wbase = wid * ECAP

    def initrow(j, _):
        for a in accs:
            for c in range(8):
                a[j, pl.ds(c * 16, 16)] = jnp.full((16,), -jnp.inf, jnp.float32)
        return ()

    lax.fori_loop(0, _NPW + 1, initrow, ())

    pltpu.sync_copy(dcnt_hbm.at[pl.ds(wid * 16, 16)], cnt_v)
    trips = cnt_v[...][0] // CSEG

    def stage(t, b):
        pltpu.make_async_copy(did_hbm.at[pl.ds(wbase + t * CSEG, CSEG)],
                              pks[b], lsems[b]).wait()
        for g in range(CSEG // 16):
            p = pks[b][pl.ds(g * 16, 16)]
            idss[b][pl.ds(g * 16, 16)] = p & ((1 << 18) - 1)
        pltpu.async_copy(m_hbm.at[idss[b]], rows[b], gsems[b])

    def compute(b):
        def group(g, _):
            dlv = pks[b][pl.ds(g * 16, 16)] >> 18
            for k in range(16):
                e = g * 16 + k
                acc = accs[k % 2]
                dl = dlv[k]
                for c in range(8):
                    r = rows[b][e, pl.ds(c * 16, 16)]
                    acc[dl, pl.ds(c * 16, 16)] = jnp.maximum(
                        acc[dl, pl.ds(c * 16, 16)], r)
            return ()

        lax.fori_loop(0, CSEG // 16, group, ())

    @pl.when(trips > 0)
    def _():
        pltpu.async_copy(did_hbm.at[pl.ds(wbase, CSEG)], pks[0], lsems[0])
        stage(0, 0)

    def chunk(t, _):
        def do(b, nb):
            @pl.when(t + 1 < trips)
            def _():
                pltpu.async_copy(
                    did_hbm.at[pl.ds(wbase + (t + 1) * CSEG, CSEG)],
                    pks[nb], lsems[nb])
                stage(t + 1, nb)
            pltpu.make_async_copy(m_hbm.at[idss[b]], rows[b], gsems[b]).wait()
            compute(b)

        @pl.when(t % 2 == 0)
        def _():
            do(0, 1)

        @pl.when(t % 2 == 1)
        def _():
            do(1, 0)

        return ()

    lax.fori_loop(0, trips, chunk, ())

    def mergerow(j, _):
        for c in range(8):
            acc0_v[j, pl.ds(c * 16, 16)] = jnp.maximum(
                acc0_v[j, pl.ds(c * 16, 16)],
                acc1_v[j, pl.ds(c * 16, 16)])
        return ()

    lax.fori_loop(0, _NPW, mergerow, ())
    pltpu.sync_copy(acc0_v.at[pl.ds(0, _NPW)],
                    aggr_hbm.at[pl.ds(wid * _NPW, _NPW)])


def _sc_segmax(m, did, dcnt):
    mesh = plsc.VectorSubcoreMesh(core_axis_name="c", subcore_axis_name="s")
    f = pl.kernel(
        _sc_segmax_body,
        mesh=mesh,
        compiler_params=_SC_PARAMS,
        out_type=jax.ShapeDtypeStruct((N, L), jnp.float32),
        scratch_types=[
            pltpu.VMEM((CSEG,), jnp.int32),
            pltpu.VMEM((CSEG,), jnp.int32),
            pltpu.VMEM((CSEG,), jnp.int32),
            pltpu.VMEM((CSEG,), jnp.int32),
            pltpu.VMEM((CSEG, L), jnp.float32),
            pltpu.VMEM((CSEG, L), jnp.float32),
            pltpu.VMEM((_NPW + 1, L), jnp.float32),
            pltpu.VMEM((_NPW + 1, L), jnp.float32),
            pltpu.VMEM((16,), jnp.int32),
            pltpu.SemaphoreType.DMA,
            pltpu.SemaphoreType.DMA,
            pltpu.SemaphoreType.DMA,
            pltpu.SemaphoreType.DMA,
        ],
    )
    return f(m, did, dcnt)


_FILLB = 16384
_RPW = N * N // _NW


def _sc_pred_body(pout_hbm, addr_hbm, spid_hbm, scnt_hbm, pred_hbm,
                  fill_v, a0_v, a1_v, p0_v, p1_v, v0_v, v1_v, cnt_v,
                  sem, la0, la1, lp0, lp1, g0, g1):
    addrs = (a0_v, a1_v)
    pids = (p0_v, p1_v)
    vals = (v0_v, v1_v)
    lasems = (la0, la1)
    lpsems = (lp0, lp1)
    gsems = (g0, g1)
    wid = lax.axis_index("s") * _NC + lax.axis_index("c")
    wbase = wid * ECAP

    def initf(j, _):
        fill_v[pl.ds(j * 16, 16)] = jnp.full((16,), _NEG, jnp.float32)
        return ()

    lax.fori_loop(0, _FILLB // 16, initf, ())

    def fill(k, _):
        pltpu.async_copy(fill_v,
                         pred_hbm.at[pl.ds(wid * _RPW + k * _FILLB, _FILLB)],
                         sem)
        return ()

    lax.fori_loop(0, _RPW // _FILLB, fill, ())

    def fill_wait(k, _):
        pltpu.make_async_copy(
            fill_v, pred_hbm.at[pl.ds(wid * _RPW + k * _FILLB, _FILLB)],
            sem).wait()
        return ()

    lax.fori_loop(0, _RPW // _FILLB, fill_wait, ())

    pltpu.sync_copy(scnt_hbm.at[pl.ds(wid * 16, 16)], cnt_v)
    trips = cnt_v[...][0] // CSEG

    def prefetch(t, b):
        pltpu.async_copy(addr_hbm.at[pl.ds(wbase + t * CSEG, CSEG)],
                         addrs[b], lasems[b])
        pltpu.async_copy(spid_hbm.at[pl.ds(wbase + t * CSEG, CSEG)],
                         pids[b], lpsems[b])

    def stage(t, b):
        pltpu.make_async_copy(addr_hbm.at[pl.ds(wbase + t * CSEG, CSEG)],
                              addrs[b], lasems[b]).wait()
        pltpu.make_async_copy(spid_hbm.at[pl.ds(wbase + t * CSEG, CSEG)],
                              pids[b], lpsems[b]).wait()
        pltpu.async_copy(pout_hbm.at[pids[b]], vals[b], gsems[b])

    @pl.when(trips > 0)
    def _():
        prefetch(0, 0)
        stage(0, 0)

    def chunk(t, _):
        def do(b, nb):
            @pl.when(t + 1 < trips)
            def _():
                prefetch(t + 1, nb)
                stage(t + 1, nb)
            pltpu.make_async_copy(pout_hbm.at[pids[b]], vals[b],
                                  gsems[b]).wait()
            pltpu.async_copy(vals[b], pred_hbm.at[addrs[b]], sem).wait()

        @pl.when(t % 2 == 0)
        def _():
            do(0, 1)

        @pl.when(t % 2 == 1)
        def _():
            do(1, 0)

        return ()

    lax.fori_loop(0, trips, chunk, ())


def _sc_pred(pout, addr, spid, scnt):
    mesh = plsc.VectorSubcoreMesh(core_axis_name="c", subcore_axis_name="s")
    f = pl.kernel(
        _sc_pred_body,
        mesh=mesh,
        compiler_params=_SC_PARAMS,
        out_type=jax.ShapeDtypeStruct((N * N + PREDPAD,), jnp.float32),
        scratch_types=[
            pltpu.VMEM((_FILLB,), jnp.float32),
            pltpu.VMEM((CSEG,), jnp.int32),
            pltpu.VMEM((CSEG,), jnp.int32),
            pltpu.VMEM((CSEG,), jnp.int32),
            pltpu.VMEM((CSEG,), jnp.int32),
            pltpu.VMEM((CSEG,), jnp.float32),
            pltpu.VMEM((CSEG,), jnp.float32),
            pltpu.VMEM((16,), jnp.int32),
            pltpu.SemaphoreType.DMA,
            pltpu.SemaphoreType.DMA,
            pltpu.SemaphoreType.DMA,
            pltpu.SemaphoreType.DMA,
            pltpu.SemaphoreType.DMA,
            pltpu.SemaphoreType.DMA,
            pltpu.SemaphoreType.DMA,
        ],
    )
    return f(pout, addr, spid, scnt)


def _prep_body(pt_ref, h_ref, w0_ref, wenc1_ref, wm1d_ref, wm1s_ref,
               enc_ref, a_ref, b_ref):
    enc = jax.nn.relu(pt_ref[...] * w0_ref[...] +
                      jnp.dot(h_ref[...], wenc1_ref[...],
                              preferred_element_type=jnp.float32))
    enc_ref[...] = enc
    a_ref[...] = jnp.dot(enc, wm1d_ref[...], preferred_element_type=jnp.float32)
    b_ref[...] = jnp.dot(enc, wm1s_ref[...], preferred_element_type=jnp.float32)


def _prep(pt, h, w0, wenc1, wm1d, wm1s):
    row = pl.BlockSpec((BN, L), lambda i: (i, 0))
    col = pl.BlockSpec((BN, 1), lambda i: (i, 0))
    full = pl.BlockSpec((L, L), lambda i: (0, 0))
    vec = pl.BlockSpec((1, L), lambda i: (0, 0))
    return pl.pallas_call(
        _prep_body,
        grid=(NBLK,),
        in_specs=[col, row, vec, full, full, full],
        out_specs=[row, row, row],
        out_shape=[jax.ShapeDtypeStruct((N, L), jnp.float32)] * 3,
    )(pt, h, w0, wenc1, wm1d, wm1s)


def _mmlp_body(a_ref, b_ref, ea_ref, w_ref, wm2_ref, o_ref):
    pre = a_ref[...] + b_ref[...] + ea_ref[...] * w_ref[...]
    q = _leaky(pre)
    o_ref[...] = _leaky(jnp.dot(q, wm2_ref[...],
                                preferred_element_type=jnp.float32))


def _mmlp(adst, bsrc, ea2, w, wm2):
    row = pl.BlockSpec((BE, L), lambda i: (i, 0))
    col = pl.BlockSpec((BE, 1), lambda i: (i, 0))
    full = pl.BlockSpec((L, L), lambda i: (0, 0))
    vec = pl.BlockSpec((1, L), lambda i: (0, 0))
    return pl.pallas_call(
        _mmlp_body,
        grid=(EBLK,),
        in_specs=[row, row, col, vec, full],
        out_specs=row,
        out_shape=jax.ShapeDtypeStruct((E, L), jnp.float32),
    )(adst, bsrc, ea2, w, wm2)


def _gru(u, gh_ref, h, w_iht_ref):
    gi = jnp.dot(u, w_iht_ref[...], preferred_element_type=jnp.float32)
    gh = gh_ref
    r = jax.nn.sigmoid(gi[:, :L] + gh[:, :L])
    z = jax.nn.sigmoid(gi[:, L:2 * L] + gh[:, L:2 * L])
    ng = jnp.tanh(gi[:, 2 * L:] + r * gh[:, 2 * L:])
    return (1.0 - z) * ng + z * h


def _update_body(enc_ref, raw_ref, h_ref, wue_ref, wua_ref, wiht_ref,
                 whht_ref, wmste_ref, wmsth_ref, hn_ref, mst_ref):
    raw = raw_ref[...]
    aggr = jnp.where(raw > -jnp.inf, raw, 0.0)
    enc = enc_ref[...]
    h = h_ref[...]
    u = _leaky(jnp.dot(enc, wue_ref[...], preferred_element_type=jnp.float32) +
               jnp.dot(aggr, wua_ref[...], preferred_element_type=jnp.float32))
    gh = jnp.dot(h, whht_ref[...], preferred_element_type=jnp.float32)
    hn = _gru(u, gh, h, wiht_ref)
    hn_ref[...] = hn
    mst_ref[...] = (jnp.dot(enc, wmste_ref[...], preferred_element_type=jnp.float32) +
                    jnp.dot(hn, wmsth_ref[...], preferred_element_type=jnp.float32))


def _update(enc, raw, h, wue, wua, wiht, whht, wmste, wmsth):
    row = pl.BlockSpec((BN, L), lambda i: (i, 0))
    full = pl.BlockSpec((L, L), lambda i: (0, 0))
    full3 = pl.BlockSpec((L, 3 * L), lambda i: (0, 0))
    cvec = pl.BlockSpec((L, 1), lambda i: (0, 0))
    col = pl.BlockSpec((BN, 1), lambda i: (i, 0))
    return pl.pallas_call(
        _update_body,
        grid=(NBLK,),
        in_specs=[row, row, row, full, full, full3, full3, cvec, cvec],
        out_specs=[row, col],
        out_shape=[jax.ShapeDtypeStruct((N, L), jnp.float32),
                   jax.ShapeDtypeStruct((N, 1), jnp.float32)],
    )(enc, raw, h, wue, wua, wiht, whht, wmste, wmsth)


def _update0_body(emax_ref, emin_ref, wm1e_ref, wm2_ref, wua_ref, wiht_ref,
                  wmsth_ref, hn_ref, mst_ref):
    g = _leaky(jnp.dot(_leaky(wm1e_ref[...]), wm2_ref[...],
                       preferred_element_type=jnp.float32))
    emax_raw = emax_ref[...]
    emin_raw = emin_ref[...]
    emax = jnp.where(emax_raw > -jnp.inf, emax_raw, 0.0)
    emin = jnp.where(emin_raw < jnp.inf, emin_raw, 0.0)
    aggr = jnp.where(g > 0, emax * g, emin * g)
    u = _leaky(jnp.dot(aggr, wua_ref[...], preferred_element_type=jnp.float32))
    gi = jnp.dot(u, wiht_ref[...], preferred_element_type=jnp.float32)
    z = jax.nn.sigmoid(gi[:, L:2 * L])
    ng = jnp.tanh(gi[:, 2 * L:])
    hn = (1.0 - z) * ng
    hn_ref[...] = hn
    mst_ref[...] = jnp.dot(hn, wmsth_ref[...], preferred_element_type=jnp.float32)


def _update0(emax, emin, wm1e, wm2, wua, wiht, wmsth):
    row = pl.BlockSpec((BN, L), lambda i: (i, 0))
    col = pl.BlockSpec((BN, 1), lambda i: (i, 0))
    full = pl.BlockSpec((L, L), lambda i: (0, 0))
    full3 = pl.BlockSpec((L, 3 * L), lambda i: (0, 0))
    vec = pl.BlockSpec((1, L), lambda i: (0, 0))
    cvec = pl.BlockSpec((L, 1), lambda i: (0, 0))
    return pl.pallas_call(
        _update0_body,
        grid=(NBLK,),
        in_specs=[col, col, vec, full, full, full3, cvec],
        out_specs=[row, col],
        out_shape=[jax.ShapeDtypeStruct((N, L), jnp.float32),
                   jax.ShapeDtypeStruct((N, 1), jnp.float32)],
    )(emax, emin, wm1e, wm2, wua, wiht, wmsth)


def _select_body(mst_ref, pt_ref, out_ref):
    mst = mst_ref[...]
    pt = pt_ref[...]
    nt = jnp.where(pt != 0, _NEG, mst)
    iota = lax.broadcasted_iota(jnp.int32, (G, N // G), 1)
    rowmax = jnp.max(nt, axis=1, keepdims=True)
    cand = jnp.where(nt == rowmax, iota, jnp.int32(2**30))
    chosen = jnp.min(cand, axis=1, keepdims=True)
    out_ref[...] = jnp.where(iota == chosen, 1.0, pt)


def _select(mst_g, pt_g):
    blk = pl.BlockSpec((G, N // G), lambda: (0, 0))
    return pl.pallas_call(
        _select_body,
        in_specs=[blk, blk],
        out_specs=blk,
        out_shape=jax.ShapeDtypeStruct((G, N // G), jnp.float32),
    )(mst_g, pt_g)


def _final_body(enc_ref, raw_ref, h_ref, wue_ref, wua_ref, wiht_ref,
                whht_ref, wp1s_ref, wp1d_ref, c_ref, d_ref):
    raw = raw_ref[...]
    aggr = jnp.where(raw > -jnp.inf, raw, 0.0)
    enc = enc_ref[...]
    h = h_ref[...]
    u = _leaky(jnp.dot(enc, wue_ref[...], preferred_element_type=jnp.float32) +
               jnp.dot(aggr, wua_ref[...], preferred_element_type=jnp.float32))
    gh = jnp.dot(h, whht_ref[...], preferred_element_type=jnp.float32)
    hn = _gru(u, gh, h, wiht_ref)
    c_ref[...] = jnp.dot(hn, wp1s_ref[...], preferred_element_type=jnp.float32)
    d_ref[...] = jnp.dot(hn, wp1d_ref[...], preferred_element_type=jnp.float32)


def _final_update(enc, raw, h, wue, wua, wiht, whht, wp1s, wp1d):
    row = pl.BlockSpec((BN, L), lambda i: (i, 0))
    full = pl.BlockSpec((L, L), lambda i: (0, 0))
    full3 = pl.BlockSpec((L, 3 * L), lambda i: (0, 0))
    return pl.pallas_call(
        _final_body,
        grid=(NBLK,),
        in_specs=[row, row, row, full, full, full3, full3, full, full],
        out_specs=[row, row],
        out_shape=[jax.ShapeDtypeStruct((N, L), jnp.float32)] * 2,
    )(enc, raw, h, wue, wua, wiht, whht, wp1s, wp1d)


def _pout_body(c_ref, d_ref, ea_ref, w_ref, wp2_ref, o_ref):
    pin = c_ref[...] + d_ref[...] + ea_ref[...] * w_ref[...]
    o_ref[...] = jnp.dot(jax.nn.relu(pin), wp2_ref[...],
                         preferred_element_type=jnp.float32)


def _pout(csrc, ddst, ea2, wp1e, wp2):
    row = pl.BlockSpec((BE, L), lambda i: (i, 0))
    col = pl.BlockSpec((BE, 1), lambda i: (i, 0))
    vec = pl.BlockSpec((1, L), lambda i: (0, 0))
    cvec = pl.BlockSpec((L, 1), lambda i: (0, 0))
    return pl.pallas_call(
        _pout_body,
        grid=(EBLK,),
        in_specs=[row, row, col, vec, cvec],
        out_specs=col,
        out_shape=jax.ShapeDtypeStruct((E, 1), jnp.float32),
    )(csrc, ddst, ea2, wp1e, wp2)


def kernel(x, edge_attr, edge_index, W_enc, W_m1, W_m2, W_u, W_ih, W_hh, W_mst, W_p1, W_p2):
    n = x.shape[0]
    steps = x.shape[1]
    src = edge_index[0]
    dst = edge_index[1]
    ea = edge_attr
    ea2 = ea[:, None]

    w_enc0 = W_enc[0:1]
    W_enc1 = W_enc[1:]
    Wm1_d = W_m1[:L]
    Wm1_s = W_m1[L:2 * L]
    wm1_e = W_m1[2 * L:2 * L + 1]
    Wu_e = W_u[:L]
    Wu_a = W_u[L:]
    W_ihT = W_ih.T
    W_hhT = W_hh.T
    Wmst_e = W_mst[:L]
    Wmst_h = W_mst[L:]
    Wp1_s = W_p1[:L]
    Wp1_d = W_p1[L:2 * L]
    wp1_e = W_p1[2 * L:2 * L + 1]

    pt = x[:, 0]

    did, dcnt, addr, spid, scnt = _sc_prep(dst, src)

    ea_max = jax.ops.segment_max(ea, dst, num_segments=n)[:, None]
    ea_min = jax.ops.segment_min(ea, dst, num_segments=n)[:, None]
    h, mst = _update0(ea_max, ea_min, wm1_e, W_m2, Wu_a, W_ihT, Wmst_h)
    pt = _select(mst.reshape(G, n // G), pt.reshape(G, n // G)).reshape(-1)

    for step in range(1, steps):
        enc, A, B = _prep(pt[:, None], h, w_enc0, W_enc1, Wm1_d, Wm1_s)
        adst, bsrc = _sc_gather2(A, B, dst, src)
        m = _mmlp(adst, bsrc, ea2, wm1_e, W_m2)
        raw = _sc_segmax(m, did, dcnt)
        if step < steps - 1:
            h, mst = _update(enc, raw, h, Wu_e, Wu_a, W_ihT, W_hhT,
                             Wmst_e, Wmst_h)
            pt = _select(mst.reshape(G, n // G),
                         pt.reshape(G, n // G)).reshape(-1)
        else:
            C, D = _final_update(enc, raw, h, Wu_e, Wu_a, W_ihT, W_hhT,
                                 Wp1_s, Wp1_d)

    csrc, ddst = _sc_gather2(C, D, src, dst)
    p_out = _pout(csrc, ddst, ea2, wp1_e, W_p2)[:, 0]
    pred_flat = _sc_pred(p_out, addr, spid, scnt)
    return pred_flat[:n * n].reshape(n, n)
